# baseline (speedup 1.0000x reference)
import functools

import jax
import jax.numpy as jnp
from jax import lax
from jax.experimental import pallas as pl
from jax.experimental.pallas import tpu as pltpu
from jax.experimental.pallas import tpu_sc as plsc

N = 10000
D = 128
H = 8
C = 16
K = 32
CUT = 5.0
NB = 3

NPAD = 10240
MARK = 10000
RB = 256
CB = 256
BD = 512
NW = 32
PW = NPAD // NW
TW = 144
NEG = -3.0e8

_HI = lax.Precision.HIGHEST


def _dot(a, b):
    return lax.dot_general(a, b, (((1,), (0,)), ((), ())), precision=_HI,
                           preferred_element_type=jnp.float32)


def _nbr_body(posr_ref, post_ref, brow_ref, bcol_ref, idx_ref):
    i = pl.program_id(0)
    r0 = i * RB
    posr = posr_ref[...]
    sqr = jnp.sum(posr * posr, axis=1, keepdims=True)
    brow = brow_ref[...]
    batch_r = bcol_ref[...]
    rowid = r0 + lax.broadcasted_iota(jnp.int32, (RB, 1), 0)

    bmin = jnp.min(batch_r)
    bmax = jnp.max(batch_r)
    col_iota = lax.broadcasted_iota(jnp.int32, (1, NPAD), 1)
    s = jnp.min(jnp.where(brow == bmin, col_iota, NPAD))
    e = jnp.max(jnp.where(brow == bmax, col_iota, -1))
    c0 = s // CB
    c1 = e // CB

    cpos = lax.broadcasted_iota(jnp.int32, (RB, K + CB), 1)

    def chunk(c, carry):
        bd, bi = carry
        posc = post_ref[:, pl.ds(c * CB, CB)]
        sqc = jnp.sum(posc * posc, axis=0, keepdims=True)
        dot = lax.dot_general(posr, posc, (((1,), (0,)), ((), ())),
                              precision=lax.Precision.DEFAULT,
                              preferred_element_type=jnp.float32)
        d2 = sqr + sqc - 2.0 * dot
        batch_c = brow_ref[:, pl.ds(c * CB, CB)]
        colid = c * CB + lax.broadcasted_iota(jnp.int32, (1, CB), 1)
        valid = ((batch_r == batch_c) & (colid != rowid)
                 & (d2 <= CUT * CUT))
        dm = jnp.where(valid, d2, jnp.inf)
        ci = jnp.where(valid, jnp.broadcast_to(colid, (RB, CB)), MARK)
        cat_d = jnp.concatenate([bd, dm], axis=1)
        cat_i = jnp.concatenate([bi, ci], axis=1)
        nd, ni = [], []
        for _ in range(K):
            m = jnp.min(cat_d, axis=1, keepdims=True)
            am = jnp.min(jnp.where(cat_d == m, cpos, K + CB),
                         axis=1, keepdims=True)
            hit = cpos == am
            vi = jnp.max(jnp.where(hit, cat_i, -1), axis=1, keepdims=True)
            nd.append(m)
            ni.append(vi)
            cat_d = jnp.where(hit, jnp.inf, cat_d)
            cat_i = jnp.where(hit, MARK, cat_i)
        return jnp.concatenate(nd, axis=1), jnp.concatenate(ni, axis=1)

    bd0 = jnp.full((RB, K), jnp.inf, jnp.float32)
    bi0 = jnp.full((RB, K), MARK, jnp.int32)
    _, bi = lax.fori_loop(c0, c1 + 1, chunk, (bd0, bi0))
    idx_ref[...] = bi


def _nbr_call(posp, post, brow, bcol):
    return pl.pallas_call(
        _nbr_body,
        grid=(NPAD // RB,),
        in_specs=[
            pl.BlockSpec((RB, 8), lambda i: (i, 0)),
            pl.BlockSpec((8, NPAD), lambda i: (0, 0)),
            pl.BlockSpec((1, NPAD), lambda i: (0, 0)),
            pl.BlockSpec((RB, 1), lambda i: (i, 0)),
        ],
        out_specs=pl.BlockSpec((RB, K), lambda i: (i, 0)),
        out_shape=jax.ShapeDtypeStruct((NPAD, K), jnp.int32),
    )(posp, post, brow, bcol)


def _table_tail(xp, aa, rowid):
    aam = jnp.where(rowid < N, aa, NEG)
    table = jnp.concatenate([xp, aam], axis=1)
    ald = jnp.concatenate([aam[:, H:], jnp.zeros_like(aam[:, H:])], axis=1)
    return table, ald


def _pre_body(z_ref, emb_ref, wc_ref, aa_ref, h_ref, t_ref, ald_ref):
    i = pl.program_id(0)
    rowid = i * BD + lax.broadcasted_iota(jnp.int32, (BD, 1), 0)
    z = z_ref[...]
    oh = (z == lax.broadcasted_iota(jnp.int32, (BD, 104), 1)).astype(jnp.float32)
    h = _dot(oh, emb_ref[...])
    xp = _dot(h, wc_ref[...])
    aa = _dot(xp, aa_ref[...])
    table, ald = _table_tail(xp, aa, rowid)
    h_ref[...] = h
    t_ref[...] = table
    ald_ref[...] = ald


def _pre_call(zcol, embp, wc0, aa0):
    return pl.pallas_call(
        _pre_body,
        grid=(NPAD // BD,),
        in_specs=[
            pl.BlockSpec((BD, 1), lambda i: (i, 0)),
            pl.BlockSpec((104, 128), lambda i: (0, 0)),
            pl.BlockSpec((128, 128), lambda i: (0, 0)),
            pl.BlockSpec((128, 16), lambda i: (0, 0)),
        ],
        out_specs=[
            pl.BlockSpec((BD, 128), lambda i: (i, 0)),
            pl.BlockSpec((BD, TW), lambda i: (i, 0)),
            pl.BlockSpec((BD, 16), lambda i: (i, 0)),
        ],
        out_shape=[
            jax.ShapeDtypeStruct((NPAD, 128), jnp.float32),
            jax.ShapeDtypeStruct((NPAD, TW), jnp.float32),
            jax.ShapeDtypeStruct((NPAD, 16), jnp.float32),
        ],
    )(zcol, embp, wc0, aa0)


def _ffn(o, h, bc, gg, be, wf, bf):
    o = o + bc
    mu = jnp.mean(o, axis=1, keepdims=True)
    xm = o - mu
    v = jnp.mean(xm * xm, axis=1, keepdims=True)
    o = xm * lax.rsqrt(v + 1e-5) * gg + be
    o = _dot(o, wf) + bf
    return 2.0 * h + o


def _mid_body(o_ref, h_ref, bc_ref, g_ref, be_ref, wf_ref, bf_ref,
              wc_ref, aa_ref, hn_ref, t_ref, ald_ref):
    i = pl.program_id(0)
    rowid = i * BD + lax.broadcasted_iota(jnp.int32, (BD, 1), 0)
    hn = _ffn(o_ref[...], h_ref[...], bc_ref[...], g_ref[...], be_ref[...],
              wf_ref[...], bf_ref[...])
    xp = _dot(hn, wc_ref[...])
    aa = _dot(xp, aa_ref[...])
    table, ald = _table_tail(xp, aa, rowid)
    hn_ref[...] = hn
    t_ref[...] = table
    ald_ref[...] = ald


def _mid_call(o, h, bc, gg, be, wf, bf, wc, aa):
    row = lambda i: (i, 0)
    fix = lambda i: (0, 0)
    return pl.pallas_call(
        _mid_body,
        grid=(NPAD // BD,),
        in_specs=[
            pl.BlockSpec((BD, 128), row),
            pl.BlockSpec((BD, 128), row),
            pl.BlockSpec((1, 128), fix),
            pl.BlockSpec((1, 128), fix),
            pl.BlockSpec((1, 128), fix),
            pl.BlockSpec((128, 128), fix),
            pl.BlockSpec((1, 128), fix),
            pl.BlockSpec((128, 128), fix),
            pl.BlockSpec((128, 16), fix),
        ],
        out_specs=[
            pl.BlockSpec((BD, 128), row),
            pl.BlockSpec((BD, TW), row),
            pl.BlockSpec((BD, 16), row),
        ],
        out_shape=[
            jax.ShapeDtypeStruct((NPAD, 128), jnp.float32),
            jax.ShapeDtypeStruct((NPAD, TW), jnp.float32),
            jax.ShapeDtypeStruct((NPAD, 16), jnp.float32),
        ],
    )(o, h, bc, gg, be, wf, bf, wc, aa)


def _post_body(o_ref, h_ref, bc_ref, g_ref, be_ref, wf_ref, bf_ref, hn_ref):
    hn_ref[...] = _ffn(o_ref[...], h_ref[...], bc_ref[...], g_ref[...],
                       be_ref[...], wf_ref[...], bf_ref[...])


def _post_call(o, h, bc, gg, be, wf, bf):
    row = lambda i: (i, 0)
    fix = lambda i: (0, 0)
    return pl.pallas_call(
        _post_body,
        grid=(NPAD // BD,),
        in_specs=[
            pl.BlockSpec((BD, 128), row),
            pl.BlockSpec((BD, 128), row),
            pl.BlockSpec((1, 128), fix),
            pl.BlockSpec((1, 128), fix),
            pl.BlockSpec((1, 128), fix),
            pl.BlockSpec((128, 128), fix),
            pl.BlockSpec((1, 128), fix),
        ],
        out_specs=pl.BlockSpec((BD, 128), row),
        out_shape=jax.ShapeDtypeStruct((NPAD, 128), jnp.float32),
    )(o, h, bc, gg, be, wf, bf)


def _lane(v, j):
    return v.at[jnp.full((16,), j, jnp.int32)].get(mode="promise_in_bounds")


def _sc_gat_body(t_hbm, idx_hbm, ald_hbm, out_hbm,
                 idx_l, ald_l, out_l, escr, rows0, rows1, sem0, sem1):
    wid = lax.axis_index("s") * 2 + lax.axis_index("c")
    base = pl.multiple_of(wid * PW, PW)
    pltpu.sync_copy(idx_hbm.at[pl.ds(pl.multiple_of(wid * (PW // 4), 8),
                                     PW // 4)], idx_l)
    pltpu.sync_copy(ald_hbm.at[pl.ds(pl.multiple_of(wid * (PW // 8), 8),
                                     PW // 8)], ald_l)

    def idx_slice(t):
        return idx_l.at[t // 4, pl.ds((t % 4) * K, K)]

    pltpu.make_async_copy(t_hbm.at[idx_slice(0)], rows0, sem0).start()
    pltpu.make_async_copy(t_hbm.at[idx_slice(1)], rows1, sem1).start()

    def compute(t, rows_ref):
        ald = ald_l[t // 8, pl.ds((t % 8) * 16, 16)]
        m = jnp.full((16,), -3e38, jnp.float32)
        for k in range(K):
            e = rows_ref[k, pl.ds(128, 16)] + ald
            e = jnp.maximum(e, 0.2 * e)
            escr[k // 8, pl.ds((k % 8) * 16, 16)] = e
            m = jnp.maximum(m, e)
        den = jnp.full((16,), 1e-16, jnp.float32)
        num = [jnp.zeros((16,), jnp.float32) for _ in range(H)]
        for k in range(K):
            ex = jnp.exp(escr[k // 8, pl.ds((k % 8) * 16, 16)] - m)
            den = den + ex
            for hi in range(H):
                num[hi] = num[hi] + _lane(ex, hi) * rows_ref[k, pl.ds(hi * C, C)]
        rden = 1.0 / den
        for hi in range(H):
            out_l[t, pl.ds(hi * C, C)] = num[hi] * _lane(rden, hi)

    def body(j, _):
        t0 = 2 * j
        pltpu.make_async_copy(t_hbm.at[idx_slice(t0)], rows0, sem0).wait()
        compute(t0, rows0)

        @pl.when(j < PW // 2 - 1)
        def _():
            pltpu.make_async_copy(t_hbm.at[idx_slice(t0 + 2)], rows0, sem0).start()

        t1 = t0 + 1
        pltpu.make_async_copy(t_hbm.at[idx_slice(t1)], rows1, sem1).wait()
        compute(t1, rows1)

        @pl.when(j < PW // 2 - 1)
        def _():
            pltpu.make_async_copy(t_hbm.at[idx_slice(t1 + 2)], rows1, sem1).start()

        return 0

    lax.fori_loop(0, PW // 2, body, 0)
    pltpu.sync_copy(out_l, out_hbm.at[pl.ds(pl.multiple_of(base, 8), PW)])


def _sc_gat(table, idx, ald):
    mesh = plsc.VectorSubcoreMesh(core_axis_name="c", subcore_axis_name="s")
    f = functools.partial(
        pl.kernel,
        mesh=mesh,
        out_type=jax.ShapeDtypeStruct((NPAD, 128), jnp.float32),
        scratch_types=[
            pltpu.VMEM((PW // 4, 128), jnp.int32),
            pltpu.VMEM((PW // 8, 128), jnp.float32),
            pltpu.VMEM((PW, 128), jnp.float32),
            pltpu.VMEM((K // 8, 128), jnp.float32),
            pltpu.VMEM((K, TW), jnp.float32),
            pltpu.VMEM((K, TW), jnp.float32),
            pltpu.SemaphoreType.DMA,
            pltpu.SemaphoreType.DMA,
        ],
        compiler_params=pltpu.CompilerParams(use_tc_tiling_on_sc=False),
    )(_sc_gat_body)
    return f(table, idx.reshape(NPAD // 4, 128), ald.reshape(NPAD // 8, 128))


def kernel(z, pos, batch, emb, Wc, asrc, adst, bc, g, be, Wf, bf):
    npad = NPAD - N
    posp = jnp.concatenate(
        [pos.astype(jnp.float32), jnp.zeros((npad, 3), jnp.float32)], axis=0)
    posp = jnp.concatenate([posp, jnp.zeros((NPAD, 5), jnp.float32)], axis=1)
    post = posp[:, :8].T
    bpad = jnp.concatenate(
        [batch.astype(jnp.int32), jnp.full((npad,), 16, jnp.int32)])
    brow = bpad.reshape(1, NPAD)
    bcol = bpad.reshape(NPAD, 1)
    zcol = jnp.concatenate(
        [z.astype(jnp.int32), jnp.zeros((npad,), jnp.int32)]).reshape(NPAD, 1)
    embp = jnp.concatenate([emb, jnp.zeros((4, D), jnp.float32)], axis=0)

    rows = jnp.arange(H * C)
    hd = rows // C
    sel = (hd[:, None] == jnp.arange(H)[None, :]).astype(jnp.float32)
    aas = [jnp.concatenate([sel * asrc[b].reshape(-1)[:, None],
                            sel * adst[b].reshape(-1)[:, None]], axis=1)
           for b in range(NB)]

    idx = _nbr_call(posp, post, brow, bcol)

    h, table, ald = _pre_call(zcol, embp, Wc[0], aas[0])
    for b in range(NB):
        o = _sc_gat(table, idx, ald)
        if b < NB - 1:
            h, table, ald = _mid_call(
                o, h, bc[b].reshape(1, -1), g[b].reshape(1, -1),
                be[b].reshape(1, -1), Wf[b], bf[b].reshape(1, -1),
                Wc[b + 1], aas[b + 1])
        else:
            h = _post_call(
                o, h, bc[b].reshape(1, -1), g[b].reshape(1, -1),
                be[b].reshape(1, -1), Wf[b], bf[b].reshape(1, -1))
    return h[:N]


# speedup vs baseline: 1.2480x; 1.2480x over previous
import functools

import jax
import jax.numpy as jnp
from jax import lax
from jax.experimental import pallas as pl
from jax.experimental.pallas import tpu as pltpu
from jax.experimental.pallas import tpu_sc as plsc

N = 10000
D = 128
H = 8
C = 16
K = 32
CUT = 5.0
NB = 3

NPAD = 10240
MARK = 10000
RB = 256
CB = 256
BD = 512
NW = 32
PW = NPAD // NW
TW = 144
NEG = -3.0e8

_HI = lax.Precision.HIGHEST


def _dot(a, b):
    return lax.dot_general(a, b, (((1,), (0,)), ((), ())), precision=_HI,
                           preferred_element_type=jnp.float32)


def _nbr_body(posr_ref, post_ref, brow_ref, bcol_ref, idx_ref):
    i = pl.program_id(0)
    r0 = i * RB
    posr = posr_ref[...]
    sqr = jnp.sum(posr * posr, axis=1, keepdims=True)
    brow = brow_ref[...]
    batch_r = bcol_ref[...]
    rowid = r0 + lax.broadcasted_iota(jnp.int32, (RB, 1), 0)

    bmin = jnp.min(batch_r)
    bmax = jnp.max(batch_r)
    col_iota = lax.broadcasted_iota(jnp.int32, (1, NPAD), 1)
    s = jnp.min(jnp.where(brow == bmin, col_iota, NPAD))
    e = jnp.max(jnp.where(brow == bmax, col_iota, -1))
    c0 = s // CB
    c1 = e // CB

    cpos = lax.broadcasted_iota(jnp.int32, (RB, K + CB), 1)

    def chunk(c, carry):
        bd, bi = carry
        posc = post_ref[:, pl.ds(c * CB, CB)]
        sqc = jnp.sum(posc * posc, axis=0, keepdims=True)
        dot = lax.dot_general(posr, posc, (((1,), (0,)), ((), ())),
                              precision=lax.Precision.DEFAULT,
                              preferred_element_type=jnp.float32)
        d2 = sqr + sqc - 2.0 * dot
        batch_c = brow_ref[:, pl.ds(c * CB, CB)]
        colid = c * CB + lax.broadcasted_iota(jnp.int32, (1, CB), 1)
        valid = ((batch_r == batch_c) & (colid != rowid)
                 & (d2 <= CUT * CUT))
        dm = jnp.where(valid, d2, jnp.inf)
        ci = jnp.where(valid, jnp.broadcast_to(colid, (RB, CB)), MARK)
        cat_d = jnp.concatenate([bd, dm], axis=1)
        cat_i = jnp.concatenate([bi, ci], axis=1)
        nd, ni = [], []
        for _ in range(K):
            m = jnp.min(cat_d, axis=1, keepdims=True)
            am = jnp.min(jnp.where(cat_d == m, cpos, K + CB),
                         axis=1, keepdims=True)
            hit = cpos == am
            vi = jnp.max(jnp.where(hit, cat_i, -1), axis=1, keepdims=True)
            nd.append(m)
            ni.append(vi)
            cat_d = jnp.where(hit, jnp.inf, cat_d)
            cat_i = jnp.where(hit, MARK, cat_i)
        return jnp.concatenate(nd, axis=1), jnp.concatenate(ni, axis=1)

    bd0 = jnp.full((RB, K), jnp.inf, jnp.float32)
    bi0 = jnp.full((RB, K), MARK, jnp.int32)
    _, bi = lax.fori_loop(c0, c1 + 1, chunk, (bd0, bi0))
    idx_ref[...] = bi


def _nbr_call(posp, post, brow, bcol):
    return pl.pallas_call(
        _nbr_body,
        grid=(NPAD // RB,),
        in_specs=[
            pl.BlockSpec((RB, 8), lambda i: (i, 0)),
            pl.BlockSpec((8, NPAD), lambda i: (0, 0)),
            pl.BlockSpec((1, NPAD), lambda i: (0, 0)),
            pl.BlockSpec((RB, 1), lambda i: (i, 0)),
        ],
        out_specs=pl.BlockSpec((RB, K), lambda i: (i, 0)),
        out_shape=jax.ShapeDtypeStruct((NPAD, K), jnp.int32),
    )(posp, post, brow, bcol)


def _prep_body(post_ref, brow_ref, p4_ref, seg_ref):
    post = post_ref[...]
    pb = post[0:3].astype(jnp.bfloat16).astype(jnp.float32)
    sq = (post[0] * post[0] + post[1] * post[1]) + post[2] * post[2]
    p4_ref[0:3, :] = pb
    p4_ref[3, :] = sq
    brow = brow_ref[...]
    iota = lax.broadcasted_iota(jnp.int32, (1, NPAD), 1)
    slo = jnp.zeros((1, NPAD), jnp.int32)
    shi = jnp.zeros((1, NPAD), jnp.int32)
    for mmol in range(16):
        mk = brow == mmol
        ms = jnp.min(jnp.where(mk, iota, NPAD))
        me = jnp.max(jnp.where(mk, iota, -1)) + 1
        slo = jnp.where(mk, ms, slo)
        shi = jnp.where(mk, me, shi)
    seg_ref[0:1, :] = slo
    seg_ref[1:2, :] = shi


def _prep_call(post, brow):
    return pl.pallas_call(
        _prep_body,
        in_specs=[
            pl.BlockSpec((8, NPAD), lambda: (0, 0)),
            pl.BlockSpec((1, NPAD), lambda: (0, 0)),
        ],
        out_specs=[
            pl.BlockSpec((4, NPAD), lambda: (0, 0)),
            pl.BlockSpec((2, NPAD), lambda: (0, 0)),
        ],
        out_shape=[
            jax.ShapeDtypeStruct((4, NPAD), jnp.float32),
            jax.ShapeDtypeStruct((2, NPAD), jnp.int32),
        ],
    )(post, brow)


def _sc_topk_body(p4_hbm, seg_hbm, idx_hbm, p_l, seg_l, out_l):
    wid = lax.axis_index("s") * 2 + lax.axis_index("c")
    base = pl.multiple_of(wid * PW, PW)
    pltpu.sync_copy(p4_hbm, p_l)
    pltpu.sync_copy(seg_hbm.at[0, pl.ds(base, PW)], seg_l.at[0])
    pltpu.sync_copy(seg_hbm.at[1, pl.ds(base, PW)], seg_l.at[1])

    lane = jnp.arange(16, dtype=jnp.int32)
    inf16 = jnp.full((16,), jnp.inf, jnp.float32)
    mark16 = jnp.full((16,), MARK, jnp.int32)

    def node(t, _):
        i = base + t
        t_al = (t // 16) * 16
        tj = jnp.full((16,), t - t_al, jnp.int32)
        sv = seg_l[0, pl.ds(t_al, 16)]
        ev = seg_l[1, pl.ds(t_al, 16)]
        s = lax.reduce_max(jnp.where(lane == tj, sv, -2**31 + 1), axes=(0,))
        e = lax.reduce_max(jnp.where(lane == tj, ev, -2**31 + 1), axes=(0,))
        i_al = (i // 16) * 16
        ij = jnp.full((16,), i - i_al, jnp.int32)
        xi = p_l[0, pl.ds(i_al, 16)].at[ij].get(mode="promise_in_bounds")
        yi = p_l[1, pl.ds(i_al, 16)].at[ij].get(mode="promise_in_bounds")
        zi = p_l[2, pl.ds(i_al, 16)].at[ij].get(mode="promise_in_bounds")
        qi = p_l[3, pl.ds(i_al, 16)].at[ij].get(mode="promise_in_bounds")
        s16 = jnp.full((16,), s, jnp.int32)
        e16 = jnp.full((16,), e, jnp.int32)
        i16 = jnp.full((16,), i, jnp.int32)
        c0 = (s // 16) * 16
        nch = (e - c0 + 15) // 16

        def chunk(c, carry):
            b0k, b0v, b1k, b1v = carry
            off = pl.multiple_of(c0 + c * 16, 8)
            xj = p_l[0, pl.ds(off, 16)]
            yj = p_l[1, pl.ds(off, 16)]
            zj = p_l[2, pl.ds(off, 16)]
            qj = p_l[3, pl.ds(off, 16)]
            dot = (xi * xj + yi * yj) + zi * zj
            d2 = (qi + qj) - 2.0 * dot
            colid = jnp.full((16,), off, jnp.int32) + lane
            ok = (colid >= s16) & (colid < e16) & (colid != i16)
            dm = jnp.where(ok, d2, inf16)
            ck, cv = plsc.sort_key_val(dm, colid)
            rk = lax.rev(ck, (0,))
            rv = lax.rev(cv, (0,))
            le = b1k <= rk
            lk = jnp.where(le, b1k, rk)
            lv = jnp.where(le, b1v, rv)
            lk, lv = plsc.sort_key_val(lk, lv)
            rk2 = lax.rev(lk, (0,))
            rv2 = lax.rev(lv, (0,))
            le2 = b0k <= rk2
            n0k = jnp.where(le2, b0k, rk2)
            n0v = jnp.where(le2, b0v, rv2)
            n1k = jnp.where(le2, rk2, b0k)
            n1v = jnp.where(le2, rv2, b0v)
            b0k, b0v = plsc.sort_key_val(n0k, n0v)
            b1k, b1v = plsc.sort_key_val(n1k, n1v)
            return b0k, b0v, b1k, b1v

        b0k, b0v, b1k, b1v = lax.fori_loop(
            0, nch, chunk, (inf16, mark16, inf16, mark16))
        out_l[t // 4, pl.ds((t % 4) * K, 16)] = b0v
        out_l[t // 4, pl.ds((t % 4) * K + 16, 16)] = b1v
        return 0

    lax.fori_loop(0, PW, node, 0)
    pltpu.sync_copy(out_l,
                    idx_hbm.at[pl.ds(pl.multiple_of(wid * (PW // 4), 8),
                                     PW // 4)])


def _sc_topk(p4, seg):
    mesh = plsc.VectorSubcoreMesh(core_axis_name="c", subcore_axis_name="s")
    f = functools.partial(
        pl.kernel,
        mesh=mesh,
        out_type=jax.ShapeDtypeStruct((NPAD // 4, 128), jnp.int32),
        scratch_types=[
            pltpu.VMEM((4, NPAD), jnp.float32),
            pltpu.VMEM((2, PW), jnp.int32),
            pltpu.VMEM((PW // 4, 128), jnp.int32),
        ],
        compiler_params=pltpu.CompilerParams(use_tc_tiling_on_sc=False,
                                             needs_layout_passes=False),
    )(_sc_topk_body)
    return f(p4, seg).reshape(NPAD, K)


def _table_tail(xp, aa, rowid):
    aam = jnp.where(rowid < N, aa, NEG)
    table = jnp.concatenate([xp, aam], axis=1)
    ald = jnp.concatenate([aam[:, H:], jnp.zeros_like(aam[:, H:])], axis=1)
    return table, ald


def _pre_body(z_ref, emb_ref, wc_ref, aa_ref, h_ref, t_ref, ald_ref):
    i = pl.program_id(0)
    rowid = i * BD + lax.broadcasted_iota(jnp.int32, (BD, 1), 0)
    z = z_ref[...]
    oh = (z == lax.broadcasted_iota(jnp.int32, (BD, 104), 1)).astype(jnp.float32)
    h = _dot(oh, emb_ref[...])
    xp = _dot(h, wc_ref[...])
    aa = _dot(xp, aa_ref[...])
    table, ald = _table_tail(xp, aa, rowid)
    h_ref[...] = h
    t_ref[...] = table
    ald_ref[...] = ald


def _pre_call(zcol, embp, wc0, aa0):
    return pl.pallas_call(
        _pre_body,
        grid=(NPAD // BD,),
        in_specs=[
            pl.BlockSpec((BD, 1), lambda i: (i, 0)),
            pl.BlockSpec((104, 128), lambda i: (0, 0)),
            pl.BlockSpec((128, 128), lambda i: (0, 0)),
            pl.BlockSpec((128, 16), lambda i: (0, 0)),
        ],
        out_specs=[
            pl.BlockSpec((BD, 128), lambda i: (i, 0)),
            pl.BlockSpec((BD, TW), lambda i: (i, 0)),
            pl.BlockSpec((BD, 16), lambda i: (i, 0)),
        ],
        out_shape=[
            jax.ShapeDtypeStruct((NPAD, 128), jnp.float32),
            jax.ShapeDtypeStruct((NPAD, TW), jnp.float32),
            jax.ShapeDtypeStruct((NPAD, 16), jnp.float32),
        ],
    )(zcol, embp, wc0, aa0)


def _ffn(o, h, bc, gg, be, wf, bf):
    o = o + bc
    mu = jnp.mean(o, axis=1, keepdims=True)
    xm = o - mu
    v = jnp.mean(xm * xm, axis=1, keepdims=True)
    o = xm * lax.rsqrt(v + 1e-5) * gg + be
    o = _dot(o, wf) + bf
    return 2.0 * h + o


def _mid_body(o_ref, h_ref, bc_ref, g_ref, be_ref, wf_ref, bf_ref,
              wc_ref, aa_ref, hn_ref, t_ref, ald_ref):
    i = pl.program_id(0)
    rowid = i * BD + lax.broadcasted_iota(jnp.int32, (BD, 1), 0)
    hn = _ffn(o_ref[...], h_ref[...], bc_ref[...], g_ref[...], be_ref[...],
              wf_ref[...], bf_ref[...])
    xp = _dot(hn, wc_ref[...])
    aa = _dot(xp, aa_ref[...])
    table, ald = _table_tail(xp, aa, rowid)
    hn_ref[...] = hn
    t_ref[...] = table
    ald_ref[...] = ald


def _mid_call(o, h, bc, gg, be, wf, bf, wc, aa):
    row = lambda i: (i, 0)
    fix = lambda i: (0, 0)
    return pl.pallas_call(
        _mid_body,
        grid=(NPAD // BD,),
        in_specs=[
            pl.BlockSpec((BD, 128), row),
            pl.BlockSpec((BD, 128), row),
            pl.BlockSpec((1, 128), fix),
            pl.BlockSpec((1, 128), fix),
            pl.BlockSpec((1, 128), fix),
            pl.BlockSpec((128, 128), fix),
            pl.BlockSpec((1, 128), fix),
            pl.BlockSpec((128, 128), fix),
            pl.BlockSpec((128, 16), fix),
        ],
        out_specs=[
            pl.BlockSpec((BD, 128), row),
            pl.BlockSpec((BD, TW), row),
            pl.BlockSpec((BD, 16), row),
        ],
        out_shape=[
            jax.ShapeDtypeStruct((NPAD, 128), jnp.float32),
            jax.ShapeDtypeStruct((NPAD, TW), jnp.float32),
            jax.ShapeDtypeStruct((NPAD, 16), jnp.float32),
        ],
    )(o, h, bc, gg, be, wf, bf, wc, aa)


def _post_body(o_ref, h_ref, bc_ref, g_ref, be_ref, wf_ref, bf_ref, hn_ref):
    hn_ref[...] = _ffn(o_ref[...], h_ref[...], bc_ref[...], g_ref[...],
                       be_ref[...], wf_ref[...], bf_ref[...])


def _post_call(o, h, bc, gg, be, wf, bf):
    row = lambda i: (i, 0)
    fix = lambda i: (0, 0)
    return pl.pallas_call(
        _post_body,
        grid=(NPAD // BD,),
        in_specs=[
            pl.BlockSpec((BD, 128), row),
            pl.BlockSpec((BD, 128), row),
            pl.BlockSpec((1, 128), fix),
            pl.BlockSpec((1, 128), fix),
            pl.BlockSpec((1, 128), fix),
            pl.BlockSpec((128, 128), fix),
            pl.BlockSpec((1, 128), fix),
        ],
        out_specs=pl.BlockSpec((BD, 128), row),
        out_shape=jax.ShapeDtypeStruct((NPAD, 128), jnp.float32),
    )(o, h, bc, gg, be, wf, bf)


def _lane(v, j):
    return v.at[jnp.full((16,), j, jnp.int32)].get(mode="promise_in_bounds")


def _sc_gat_body(t_hbm, idx_hbm, ald_hbm, out_hbm,
                 idx_l, ald_l, out_l, escr, rows0, rows1, sem0, sem1):
    wid = lax.axis_index("s") * 2 + lax.axis_index("c")
    base = pl.multiple_of(wid * PW, PW)
    pltpu.sync_copy(idx_hbm.at[pl.ds(pl.multiple_of(wid * (PW // 4), 8),
                                     PW // 4)], idx_l)
    pltpu.sync_copy(ald_hbm.at[pl.ds(pl.multiple_of(wid * (PW // 8), 8),
                                     PW // 8)], ald_l)

    def idx_slice(t):
        return idx_l.at[t // 4, pl.ds((t % 4) * K, K)]

    pltpu.make_async_copy(t_hbm.at[idx_slice(0)], rows0, sem0).start()
    pltpu.make_async_copy(t_hbm.at[idx_slice(1)], rows1, sem1).start()

    def compute(t, rows_ref):
        ald = ald_l[t // 8, pl.ds((t % 8) * 16, 16)]
        m = jnp.full((16,), -3e38, jnp.float32)
        for k in range(K):
            e = rows_ref[k, pl.ds(128, 16)] + ald
            e = jnp.maximum(e, 0.2 * e)
            escr[k // 8, pl.ds((k % 8) * 16, 16)] = e
            m = jnp.maximum(m, e)
        den = jnp.full((16,), 1e-16, jnp.float32)
        num = [jnp.zeros((16,), jnp.float32) for _ in range(H)]
        for k in range(K):
            ex = jnp.exp(escr[k // 8, pl.ds((k % 8) * 16, 16)] - m)
            den = den + ex
            for hi in range(H):
                num[hi] = num[hi] + _lane(ex, hi) * rows_ref[k, pl.ds(hi * C, C)]
        rden = 1.0 / den
        for hi in range(H):
            out_l[t, pl.ds(hi * C, C)] = num[hi] * _lane(rden, hi)

    def body(j, _):
        t0 = 2 * j
        pltpu.make_async_copy(t_hbm.at[idx_slice(t0)], rows0, sem0).wait()
        compute(t0, rows0)

        @pl.when(j < PW // 2 - 1)
        def _():
            pltpu.make_async_copy(t_hbm.at[idx_slice(t0 + 2)], rows0, sem0).start()

        t1 = t0 + 1
        pltpu.make_async_copy(t_hbm.at[idx_slice(t1)], rows1, sem1).wait()
        compute(t1, rows1)

        @pl.when(j < PW // 2 - 1)
        def _():
            pltpu.make_async_copy(t_hbm.at[idx_slice(t1 + 2)], rows1, sem1).start()

        return 0

    lax.fori_loop(0, PW // 2, body, 0)
    pltpu.sync_copy(out_l, out_hbm.at[pl.ds(pl.multiple_of(base, 8), PW)])


def _sc_gat(table, idx, ald):
    mesh = plsc.VectorSubcoreMesh(core_axis_name="c", subcore_axis_name="s")
    f = functools.partial(
        pl.kernel,
        mesh=mesh,
        out_type=jax.ShapeDtypeStruct((NPAD, 128), jnp.float32),
        scratch_types=[
            pltpu.VMEM((PW // 4, 128), jnp.int32),
            pltpu.VMEM((PW // 8, 128), jnp.float32),
            pltpu.VMEM((PW, 128), jnp.float32),
            pltpu.VMEM((K // 8, 128), jnp.float32),
            pltpu.VMEM((K, TW), jnp.float32),
            pltpu.VMEM((K, TW), jnp.float32),
            pltpu.SemaphoreType.DMA,
            pltpu.SemaphoreType.DMA,
        ],
        compiler_params=pltpu.CompilerParams(use_tc_tiling_on_sc=False),
    )(_sc_gat_body)
    return f(table, idx.reshape(NPAD // 4, 128), ald.reshape(NPAD // 8, 128))


def kernel(z, pos, batch, emb, Wc, asrc, adst, bc, g, be, Wf, bf):
    npad = NPAD - N
    posp = jnp.concatenate(
        [pos.astype(jnp.float32), jnp.zeros((npad, 3), jnp.float32)], axis=0)
    posp = jnp.concatenate([posp, jnp.zeros((NPAD, 5), jnp.float32)], axis=1)
    post = posp[:, :8].T
    bpad = jnp.concatenate(
        [batch.astype(jnp.int32), jnp.full((npad,), 16, jnp.int32)])
    brow = bpad.reshape(1, NPAD)
    bcol = bpad.reshape(NPAD, 1)
    zcol = jnp.concatenate(
        [z.astype(jnp.int32), jnp.zeros((npad,), jnp.int32)]).reshape(NPAD, 1)
    embp = jnp.concatenate([emb, jnp.zeros((4, D), jnp.float32)], axis=0)

    rows = jnp.arange(H * C)
    hd = rows // C
    sel = (hd[:, None] == jnp.arange(H)[None, :]).astype(jnp.float32)
    aas = [jnp.concatenate([sel * asrc[b].reshape(-1)[:, None],
                            sel * adst[b].reshape(-1)[:, None]], axis=1)
           for b in range(NB)]

    p4, seg = _prep_call(post, brow)
    idx = _sc_topk(p4, seg)

    h, table, ald = _pre_call(zcol, embp, Wc[0], aas[0])
    for b in range(NB):
        o = _sc_gat(table, idx, ald)
        if b < NB - 1:
            h, table, ald = _mid_call(
                o, h, bc[b].reshape(1, -1), g[b].reshape(1, -1),
                be[b].reshape(1, -1), Wf[b], bf[b].reshape(1, -1),
                Wc[b + 1], aas[b + 1])
        else:
            h = _post_call(
                o, h, bc[b].reshape(1, -1), g[b].reshape(1, -1),
                be[b].reshape(1, -1), Wf[b], bf[b].reshape(1, -1))
    return h[:N]


# speedup vs baseline: 1.3503x; 1.0819x over previous
import functools

import jax
import jax.numpy as jnp
from jax import lax
from jax.experimental import pallas as pl
from jax.experimental.pallas import tpu as pltpu
from jax.experimental.pallas import tpu_sc as plsc

N = 10000
D = 128
H = 8
C = 16
K = 32
CUT = 5.0
NB = 3

NPAD = 10240
MARK = 10000
RB = 256
CB = 256
BD = 512
NW = 32
PW = NPAD // NW
TW = 144
NEG = -3.0e8

_HI = lax.Precision.HIGHEST


def _dot(a, b):
    return lax.dot_general(a, b, (((1,), (0,)), ((), ())), precision=_HI,
                           preferred_element_type=jnp.float32)


def _nbr_body(posr_ref, post_ref, brow_ref, bcol_ref, idx_ref):
    i = pl.program_id(0)
    r0 = i * RB
    posr = posr_ref[...]
    sqr = jnp.sum(posr * posr, axis=1, keepdims=True)
    brow = brow_ref[...]
    batch_r = bcol_ref[...]
    rowid = r0 + lax.broadcasted_iota(jnp.int32, (RB, 1), 0)

    bmin = jnp.min(batch_r)
    bmax = jnp.max(batch_r)
    col_iota = lax.broadcasted_iota(jnp.int32, (1, NPAD), 1)
    s = jnp.min(jnp.where(brow == bmin, col_iota, NPAD))
    e = jnp.max(jnp.where(brow == bmax, col_iota, -1))
    c0 = s // CB
    c1 = e // CB

    cpos = lax.broadcasted_iota(jnp.int32, (RB, K + CB), 1)

    def chunk(c, carry):
        bd, bi = carry
        posc = post_ref[:, pl.ds(c * CB, CB)]
        sqc = jnp.sum(posc * posc, axis=0, keepdims=True)
        dot = lax.dot_general(posr, posc, (((1,), (0,)), ((), ())),
                              precision=lax.Precision.DEFAULT,
                              preferred_element_type=jnp.float32)
        d2 = sqr + sqc - 2.0 * dot
        batch_c = brow_ref[:, pl.ds(c * CB, CB)]
        colid = c * CB + lax.broadcasted_iota(jnp.int32, (1, CB), 1)
        valid = ((batch_r == batch_c) & (colid != rowid)
                 & (d2 <= CUT * CUT))
        dm = jnp.where(valid, d2, jnp.inf)
        ci = jnp.where(valid, jnp.broadcast_to(colid, (RB, CB)), MARK)
        cat_d = jnp.concatenate([bd, dm], axis=1)
        cat_i = jnp.concatenate([bi, ci], axis=1)
        nd, ni = [], []
        for _ in range(K):
            m = jnp.min(cat_d, axis=1, keepdims=True)
            am = jnp.min(jnp.where(cat_d == m, cpos, K + CB),
                         axis=1, keepdims=True)
            hit = cpos == am
            vi = jnp.max(jnp.where(hit, cat_i, -1), axis=1, keepdims=True)
            nd.append(m)
            ni.append(vi)
            cat_d = jnp.where(hit, jnp.inf, cat_d)
            cat_i = jnp.where(hit, MARK, cat_i)
        return jnp.concatenate(nd, axis=1), jnp.concatenate(ni, axis=1)

    bd0 = jnp.full((RB, K), jnp.inf, jnp.float32)
    bi0 = jnp.full((RB, K), MARK, jnp.int32)
    _, bi = lax.fori_loop(c0, c1 + 1, chunk, (bd0, bi0))
    idx_ref[...] = bi


def _nbr_call(posp, post, brow, bcol):
    return pl.pallas_call(
        _nbr_body,
        grid=(NPAD // RB,),
        in_specs=[
            pl.BlockSpec((RB, 8), lambda i: (i, 0)),
            pl.BlockSpec((8, NPAD), lambda i: (0, 0)),
            pl.BlockSpec((1, NPAD), lambda i: (0, 0)),
            pl.BlockSpec((RB, 1), lambda i: (i, 0)),
        ],
        out_specs=pl.BlockSpec((RB, K), lambda i: (i, 0)),
        out_shape=jax.ShapeDtypeStruct((NPAD, K), jnp.int32),
    )(posp, post, brow, bcol)


def _prep_body(post_ref, brow_ref, p4_ref, seg_ref):
    post = post_ref[...]
    pb = post[0:3].astype(jnp.bfloat16).astype(jnp.float32)
    sq = (post[0] * post[0] + post[1] * post[1]) + post[2] * post[2]
    p4_ref[0:3, :] = pb
    p4_ref[3, :] = sq
    brow = brow_ref[...]
    iota = lax.broadcasted_iota(jnp.int32, (1, NPAD), 1)
    slo = jnp.zeros((1, NPAD), jnp.int32)
    shi = jnp.zeros((1, NPAD), jnp.int32)
    for mmol in range(16):
        mk = brow == mmol
        ms = jnp.min(jnp.where(mk, iota, NPAD))
        me = jnp.max(jnp.where(mk, iota, -1)) + 1
        slo = jnp.where(mk, ms, slo)
        shi = jnp.where(mk, me, shi)
    seg_ref[0:1, :] = slo
    seg_ref[1:2, :] = shi


def _prep_call(post, brow):
    return pl.pallas_call(
        _prep_body,
        in_specs=[
            pl.BlockSpec((8, NPAD), lambda: (0, 0)),
            pl.BlockSpec((1, NPAD), lambda: (0, 0)),
        ],
        out_specs=[
            pl.BlockSpec((4, NPAD), lambda: (0, 0)),
            pl.BlockSpec((2, NPAD), lambda: (0, 0)),
        ],
        out_shape=[
            jax.ShapeDtypeStruct((4, NPAD), jnp.float32),
            jax.ShapeDtypeStruct((2, NPAD), jnp.int32),
        ],
    )(post, brow)


def _sc_topk_body(p4_hbm, seg_hbm, idx_hbm, p_l, seg_l, out_l):
    wid = lax.axis_index("s") * 2 + lax.axis_index("c")
    base = pl.multiple_of(wid * PW, PW)
    pltpu.sync_copy(p4_hbm, p_l)
    pltpu.sync_copy(seg_hbm.at[0, pl.ds(base, PW)], seg_l.at[0])
    pltpu.sync_copy(seg_hbm.at[1, pl.ds(base, PW)], seg_l.at[1])

    lane = jnp.arange(16, dtype=jnp.int32)
    inf16 = jnp.full((16,), jnp.inf, jnp.float32)
    mark16 = jnp.full((16,), MARK, jnp.int32)

    def setup(t):
        i = base + t
        t_al = (t // 16) * 16
        tj = jnp.full((16,), t - t_al, jnp.int32)
        sv = seg_l[0, pl.ds(t_al, 16)]
        ev = seg_l[1, pl.ds(t_al, 16)]
        s = lax.reduce_max(jnp.where(lane == tj, sv, -2**31 + 1), axes=(0,))
        e = lax.reduce_max(jnp.where(lane == tj, ev, -2**31 + 1), axes=(0,))
        i_al = (i // 16) * 16
        ij = jnp.full((16,), i - i_al, jnp.int32)
        xi = p_l[0, pl.ds(i_al, 16)].at[ij].get(mode="promise_in_bounds")
        yi = p_l[1, pl.ds(i_al, 16)].at[ij].get(mode="promise_in_bounds")
        zi = p_l[2, pl.ds(i_al, 16)].at[ij].get(mode="promise_in_bounds")
        qi = p_l[3, pl.ds(i_al, 16)].at[ij].get(mode="promise_in_bounds")
        s16 = jnp.full((16,), s, jnp.int32)
        e16 = jnp.full((16,), e, jnp.int32)
        i16 = jnp.full((16,), i, jnp.int32)
        c0 = (s // 16) * 16
        nch = (e - c0 + 15) // 16
        return (xi, yi, zi, qi, s16, e16, i16, c0), nch

    def step(c, st, best):
        xi, yi, zi, qi, s16, e16, i16, c0 = st
        b0k, b0v, b1k, b1v = best
        off = pl.multiple_of(c0 + c * 16, 8)
        xj = p_l[0, pl.ds(off, 16)]
        yj = p_l[1, pl.ds(off, 16)]
        zj = p_l[2, pl.ds(off, 16)]
        qj = p_l[3, pl.ds(off, 16)]
        dot = (xi * xj + yi * yj) + zi * zj
        d2 = (qi + qj) - 2.0 * dot
        colid = jnp.full((16,), off, jnp.int32) + lane
        ok = (colid >= s16) & (colid < e16) & (colid != i16)
        dm = jnp.where(ok, d2, inf16)
        ck, cv = plsc.sort_key_val(dm, colid)
        rk = lax.rev(ck, (0,))
        rv = lax.rev(cv, (0,))
        le = b1k <= rk
        lk = jnp.where(le, b1k, rk)
        lv = jnp.where(le, b1v, rv)
        lk, lv = plsc.sort_key_val(lk, lv)
        rk2 = lax.rev(lk, (0,))
        rv2 = lax.rev(lv, (0,))
        le2 = b0k <= rk2
        n0k = jnp.where(le2, b0k, rk2)
        n0v = jnp.where(le2, b0v, rv2)
        n1k = jnp.where(le2, rk2, b0k)
        n1v = jnp.where(le2, rv2, b0v)
        b0k, b0v = plsc.sort_key_val(n0k, n0v)
        b1k, b1v = plsc.sort_key_val(n1k, n1v)
        return b0k, b0v, b1k, b1v

    def emit(t, best):
        out_l[t // 4, pl.ds((t % 4) * K, 16)] = best[1]
        out_l[t // 4, pl.ds((t % 4) * K + 16, 16)] = best[3]

    def node_pair(j, _):
        t0 = 2 * j
        t1 = 2 * j + 1
        st0, nch0 = setup(t0)
        st1, nch1 = setup(t1)
        init = (inf16, mark16, inf16, mark16)

        def chunk(c, carry):
            a, b = carry
            return step(c, st0, a), step(c, st1, b)

        a, b = lax.fori_loop(0, jnp.maximum(nch0, nch1), chunk, (init, init))
        emit(t0, a)
        emit(t1, b)
        return 0

    lax.fori_loop(0, PW // 2, node_pair, 0)
    pltpu.sync_copy(out_l,
                    idx_hbm.at[pl.ds(pl.multiple_of(wid * (PW // 4), 8),
                                     PW // 4)])


def _sc_topk(p4, seg):
    mesh = plsc.VectorSubcoreMesh(core_axis_name="c", subcore_axis_name="s")
    f = functools.partial(
        pl.kernel,
        mesh=mesh,
        out_type=jax.ShapeDtypeStruct((NPAD // 4, 128), jnp.int32),
        scratch_types=[
            pltpu.VMEM((4, NPAD), jnp.float32),
            pltpu.VMEM((2, PW), jnp.int32),
            pltpu.VMEM((PW // 4, 128), jnp.int32),
        ],
        compiler_params=pltpu.CompilerParams(use_tc_tiling_on_sc=False,
                                             needs_layout_passes=False),
    )(_sc_topk_body)
    return f(p4, seg).reshape(NPAD, K)


def _table_tail(xp, aa, rowid):
    aam = jnp.where(rowid < N, aa, NEG)
    table = jnp.concatenate([xp, aam], axis=1)
    ald = jnp.concatenate([aam[:, H:], jnp.zeros_like(aam[:, H:])], axis=1)
    return table, ald


def _pre_body(z_ref, emb_ref, wc_ref, aa_ref, h_ref, t_ref, ald_ref):
    i = pl.program_id(0)
    rowid = i * BD + lax.broadcasted_iota(jnp.int32, (BD, 1), 0)
    z = z_ref[...]
    oh = (z == lax.broadcasted_iota(jnp.int32, (BD, 104), 1)).astype(jnp.float32)
    h = _dot(oh, emb_ref[...])
    xp = _dot(h, wc_ref[...])
    aa = _dot(xp, aa_ref[...])
    table, ald = _table_tail(xp, aa, rowid)
    h_ref[...] = h
    t_ref[...] = table
    ald_ref[...] = ald


def _pre_call(zcol, embp, wc0, aa0):
    return pl.pallas_call(
        _pre_body,
        grid=(NPAD // BD,),
        in_specs=[
            pl.BlockSpec((BD, 1), lambda i: (i, 0)),
            pl.BlockSpec((104, 128), lambda i: (0, 0)),
            pl.BlockSpec((128, 128), lambda i: (0, 0)),
            pl.BlockSpec((128, 16), lambda i: (0, 0)),
        ],
        out_specs=[
            pl.BlockSpec((BD, 128), lambda i: (i, 0)),
            pl.BlockSpec((BD, TW), lambda i: (i, 0)),
            pl.BlockSpec((BD, 16), lambda i: (i, 0)),
        ],
        out_shape=[
            jax.ShapeDtypeStruct((NPAD, 128), jnp.float32),
            jax.ShapeDtypeStruct((NPAD, TW), jnp.float32),
            jax.ShapeDtypeStruct((NPAD, 16), jnp.float32),
        ],
    )(zcol, embp, wc0, aa0)


def _ffn(o, h, bc, gg, be, wf, bf):
    o = o + bc
    mu = jnp.mean(o, axis=1, keepdims=True)
    xm = o - mu
    v = jnp.mean(xm * xm, axis=1, keepdims=True)
    o = xm * lax.rsqrt(v + 1e-5) * gg + be
    o = _dot(o, wf) + bf
    return 2.0 * h + o


def _mid_body(o_ref, h_ref, bc_ref, g_ref, be_ref, wf_ref, bf_ref,
              wc_ref, aa_ref, hn_ref, t_ref, ald_ref):
    i = pl.program_id(0)
    rowid = i * BD + lax.broadcasted_iota(jnp.int32, (BD, 1), 0)
    hn = _ffn(o_ref[...], h_ref[...], bc_ref[...], g_ref[...], be_ref[...],
              wf_ref[...], bf_ref[...])
    xp = _dot(hn, wc_ref[...])
    aa = _dot(xp, aa_ref[...])
    table, ald = _table_tail(xp, aa, rowid)
    hn_ref[...] = hn
    t_ref[...] = table
    ald_ref[...] = ald


def _mid_call(o, h, bc, gg, be, wf, bf, wc, aa):
    row = lambda i: (i, 0)
    fix = lambda i: (0, 0)
    return pl.pallas_call(
        _mid_body,
        grid=(NPAD // BD,),
        in_specs=[
            pl.BlockSpec((BD, 128), row),
            pl.BlockSpec((BD, 128), row),
            pl.BlockSpec((1, 128), fix),
            pl.BlockSpec((1, 128), fix),
            pl.BlockSpec((1, 128), fix),
            pl.BlockSpec((128, 128), fix),
            pl.BlockSpec((1, 128), fix),
            pl.BlockSpec((128, 128), fix),
            pl.BlockSpec((128, 16), fix),
        ],
        out_specs=[
            pl.BlockSpec((BD, 128), row),
            pl.BlockSpec((BD, TW), row),
            pl.BlockSpec((BD, 16), row),
        ],
        out_shape=[
            jax.ShapeDtypeStruct((NPAD, 128), jnp.float32),
            jax.ShapeDtypeStruct((NPAD, TW), jnp.float32),
            jax.ShapeDtypeStruct((NPAD, 16), jnp.float32),
        ],
    )(o, h, bc, gg, be, wf, bf, wc, aa)


def _post_body(o_ref, h_ref, bc_ref, g_ref, be_ref, wf_ref, bf_ref, hn_ref):
    hn_ref[...] = _ffn(o_ref[...], h_ref[...], bc_ref[...], g_ref[...],
                       be_ref[...], wf_ref[...], bf_ref[...])


def _post_call(o, h, bc, gg, be, wf, bf):
    row = lambda i: (i, 0)
    fix = lambda i: (0, 0)
    return pl.pallas_call(
        _post_body,
        grid=(NPAD // BD,),
        in_specs=[
            pl.BlockSpec((BD, 128), row),
            pl.BlockSpec((BD, 128), row),
            pl.BlockSpec((1, 128), fix),
            pl.BlockSpec((1, 128), fix),
            pl.BlockSpec((1, 128), fix),
            pl.BlockSpec((128, 128), fix),
            pl.BlockSpec((1, 128), fix),
        ],
        out_specs=pl.BlockSpec((BD, 128), row),
        out_shape=jax.ShapeDtypeStruct((NPAD, 128), jnp.float32),
    )(o, h, bc, gg, be, wf, bf)


def _lane(v, j):
    return v.at[jnp.full((16,), j, jnp.int32)].get(mode="promise_in_bounds")


def _sc_gat_body(t_hbm, idx_hbm, ald_hbm, out_hbm,
                 idx_l, ald_l, out_l, escr, rows0, rows1, sem0, sem1):
    wid = lax.axis_index("s") * 2 + lax.axis_index("c")
    base = pl.multiple_of(wid * PW, PW)
    pltpu.sync_copy(idx_hbm.at[pl.ds(pl.multiple_of(wid * (PW // 4), 8),
                                     PW // 4)], idx_l)
    pltpu.sync_copy(ald_hbm.at[pl.ds(pl.multiple_of(wid * (PW // 8), 8),
                                     PW // 8)], ald_l)

    def idx_slice(t):
        return idx_l.at[t // 4, pl.ds((t % 4) * K, K)]

    pltpu.make_async_copy(t_hbm.at[idx_slice(0)], rows0, sem0).start()
    pltpu.make_async_copy(t_hbm.at[idx_slice(1)], rows1, sem1).start()

    def compute(t, rows_ref):
        ald = ald_l[t // 8, pl.ds((t % 8) * 16, 16)]
        m = jnp.full((16,), -3e38, jnp.float32)
        for k in range(K):
            e = rows_ref[k, pl.ds(128, 16)] + ald
            e = jnp.maximum(e, 0.2 * e)
            escr[k // 8, pl.ds((k % 8) * 16, 16)] = e
            m = jnp.maximum(m, e)
        den = jnp.full((16,), 1e-16, jnp.float32)
        num = [jnp.zeros((16,), jnp.float32) for _ in range(H)]
        for k in range(K):
            ex = jnp.exp(escr[k // 8, pl.ds((k % 8) * 16, 16)] - m)
            den = den + ex
            for hi in range(H):
                num[hi] = num[hi] + _lane(ex, hi) * rows_ref[k, pl.ds(hi * C, C)]
        rden = 1.0 / den
        for hi in range(H):
            out_l[t, pl.ds(hi * C, C)] = num[hi] * _lane(rden, hi)

    def body(j, _):
        t0 = 2 * j
        pltpu.make_async_copy(t_hbm.at[idx_slice(t0)], rows0, sem0).wait()
        compute(t0, rows0)

        @pl.when(j < PW // 2 - 1)
        def _():
            pltpu.make_async_copy(t_hbm.at[idx_slice(t0 + 2)], rows0, sem0).start()

        t1 = t0 + 1
        pltpu.make_async_copy(t_hbm.at[idx_slice(t1)], rows1, sem1).wait()
        compute(t1, rows1)

        @pl.when(j < PW // 2 - 1)
        def _():
            pltpu.make_async_copy(t_hbm.at[idx_slice(t1 + 2)], rows1, sem1).start()

        return 0

    lax.fori_loop(0, PW // 2, body, 0)
    pltpu.sync_copy(out_l, out_hbm.at[pl.ds(pl.multiple_of(base, 8), PW)])


def _sc_gat(table, idx, ald):
    mesh = plsc.VectorSubcoreMesh(core_axis_name="c", subcore_axis_name="s")
    f = functools.partial(
        pl.kernel,
        mesh=mesh,
        out_type=jax.ShapeDtypeStruct((NPAD, 128), jnp.float32),
        scratch_types=[
            pltpu.VMEM((PW // 4, 128), jnp.int32),
            pltpu.VMEM((PW // 8, 128), jnp.float32),
            pltpu.VMEM((PW, 128), jnp.float32),
            pltpu.VMEM((K // 8, 128), jnp.float32),
            pltpu.VMEM((K, TW), jnp.float32),
            pltpu.VMEM((K, TW), jnp.float32),
            pltpu.SemaphoreType.DMA,
            pltpu.SemaphoreType.DMA,
        ],
        compiler_params=pltpu.CompilerParams(use_tc_tiling_on_sc=False),
    )(_sc_gat_body)
    return f(table, idx.reshape(NPAD // 4, 128), ald.reshape(NPAD // 8, 128))


def kernel(z, pos, batch, emb, Wc, asrc, adst, bc, g, be, Wf, bf):
    npad = NPAD - N
    posp = jnp.concatenate(
        [pos.astype(jnp.float32), jnp.zeros((npad, 3), jnp.float32)], axis=0)
    posp = jnp.concatenate([posp, jnp.zeros((NPAD, 5), jnp.float32)], axis=1)
    post = posp[:, :8].T
    bpad = jnp.concatenate(
        [batch.astype(jnp.int32), jnp.full((npad,), 16, jnp.int32)])
    brow = bpad.reshape(1, NPAD)
    bcol = bpad.reshape(NPAD, 1)
    zcol = jnp.concatenate(
        [z.astype(jnp.int32), jnp.zeros((npad,), jnp.int32)]).reshape(NPAD, 1)
    embp = jnp.concatenate([emb, jnp.zeros((4, D), jnp.float32)], axis=0)

    rows = jnp.arange(H * C)
    hd = rows // C
    sel = (hd[:, None] == jnp.arange(H)[None, :]).astype(jnp.float32)
    aas = [jnp.concatenate([sel * asrc[b].reshape(-1)[:, None],
                            sel * adst[b].reshape(-1)[:, None]], axis=1)
           for b in range(NB)]

    p4, seg = _prep_call(post, brow)
    idx = _sc_topk(p4, seg)

    h, table, ald = _pre_call(zcol, embp, Wc[0], aas[0])
    for b in range(NB):
        o = _sc_gat(table, idx, ald)
        if b < NB - 1:
            h, table, ald = _mid_call(
                o, h, bc[b].reshape(1, -1), g[b].reshape(1, -1),
                be[b].reshape(1, -1), Wf[b], bf[b].reshape(1, -1),
                Wc[b + 1], aas[b + 1])
        else:
            h = _post_call(
                o, h, bc[b].reshape(1, -1), g[b].reshape(1, -1),
                be[b].reshape(1, -1), Wf[b], bf[b].reshape(1, -1))
    return h[:N]


# speedup vs baseline: 1.6993x; 1.2585x over previous
import functools

import jax
import jax.numpy as jnp
from jax import lax
from jax.experimental import pallas as pl
from jax.experimental.pallas import tpu as pltpu
from jax.experimental.pallas import tpu_sc as plsc

N = 10000
D = 128
H = 8
C = 16
K = 32
CUT = 5.0
NB = 3

NPAD = 10240
MARK = 10000
RB = 256
CB = 256
BD = 512
NW = 32
PW = NPAD // NW
TW = 144
NEG = -3.0e8

_HI = lax.Precision.HIGHEST


def _dot(a, b):
    return lax.dot_general(a, b, (((1,), (0,)), ((), ())), precision=_HI,
                           preferred_element_type=jnp.float32)


def _nbr_body(posr_ref, post_ref, brow_ref, bcol_ref, idx_ref):
    i = pl.program_id(0)
    r0 = (i + _OFF) * RB
    posr = posr_ref[...]
    sqr = jnp.sum(posr * posr, axis=1, keepdims=True)
    brow = brow_ref[...]
    batch_r = bcol_ref[...]
    rowid = r0 + lax.broadcasted_iota(jnp.int32, (RB, 1), 0)

    bmin = jnp.min(batch_r)
    bmax = jnp.max(batch_r)
    col_iota = lax.broadcasted_iota(jnp.int32, (1, NPAD), 1)
    s = jnp.min(jnp.where(brow == bmin, col_iota, NPAD))
    e = jnp.max(jnp.where(brow == bmax, col_iota, -1))
    c0 = s // CB
    c1 = e // CB

    cpos = lax.broadcasted_iota(jnp.int32, (RB, K + CB), 1)

    def chunk(c, carry):
        bd, bi = carry
        posc = post_ref[:, pl.ds(c * CB, CB)]
        sqc = jnp.sum(posc * posc, axis=0, keepdims=True)
        dot = lax.dot_general(posr, posc, (((1,), (0,)), ((), ())),
                              precision=lax.Precision.DEFAULT,
                              preferred_element_type=jnp.float32)
        d2 = sqr + sqc - 2.0 * dot
        batch_c = brow_ref[:, pl.ds(c * CB, CB)]
        colid = c * CB + lax.broadcasted_iota(jnp.int32, (1, CB), 1)
        valid = ((batch_r == batch_c) & (colid != rowid)
                 & (d2 <= CUT * CUT))
        dm = jnp.where(valid, d2, jnp.inf)
        ci = jnp.where(valid, jnp.broadcast_to(colid, (RB, CB)), MARK)
        cat_d = jnp.concatenate([bd, dm], axis=1)
        cat_i = jnp.concatenate([bi, ci], axis=1)
        nd, ni = [], []
        for _ in range(K):
            m = jnp.min(cat_d, axis=1, keepdims=True)
            am = jnp.min(jnp.where(cat_d == m, cpos, K + CB),
                         axis=1, keepdims=True)
            hit = cpos == am
            vi = jnp.max(jnp.where(hit, cat_i, -1), axis=1, keepdims=True)
            nd.append(m)
            ni.append(vi)
            cat_d = jnp.where(hit, jnp.inf, cat_d)
            cat_i = jnp.where(hit, MARK, cat_i)
        return jnp.concatenate(nd, axis=1), jnp.concatenate(ni, axis=1)

    bd0 = jnp.full((RB, K), jnp.inf, jnp.float32)
    bi0 = jnp.full((RB, K), MARK, jnp.int32)
    _, bi = lax.fori_loop(c0, c1 + 1, chunk, (bd0, bi0))
    idx_ref[...] = bi


NSC = 6144
PWS = NSC // NW
_OFF = NSC // RB


def _nbr_call(posp, post, brow, bcol):
    return pl.pallas_call(
        _nbr_body,
        grid=((NPAD - NSC) // RB,),
        in_specs=[
            pl.BlockSpec((RB, 8), lambda i: (i + _OFF, 0)),
            pl.BlockSpec((8, NPAD), lambda i: (0, 0)),
            pl.BlockSpec((1, NPAD), lambda i: (0, 0)),
            pl.BlockSpec((RB, 1), lambda i: (i + _OFF, 0)),
        ],
        out_specs=pl.BlockSpec((RB, K), lambda i: (i, 0)),
        out_shape=jax.ShapeDtypeStruct((NPAD - NSC, K), jnp.int32),
    )(posp, post, brow, bcol)


def _prep_body(post_ref, brow_ref, p4_ref, seg_ref):
    post = post_ref[...]
    pb = post[0:3].astype(jnp.bfloat16).astype(jnp.float32)
    sq = (post[0] * post[0] + post[1] * post[1]) + post[2] * post[2]
    p4_ref[0:3, :] = pb
    p4_ref[3, :] = sq
    brow = brow_ref[...]
    iota = lax.broadcasted_iota(jnp.int32, (1, NPAD), 1)
    slo = jnp.zeros((1, NPAD), jnp.int32)
    shi = jnp.zeros((1, NPAD), jnp.int32)
    for mmol in range(16):
        mk = brow == mmol
        ms = jnp.min(jnp.where(mk, iota, NPAD))
        me = jnp.max(jnp.where(mk, iota, -1)) + 1
        slo = jnp.where(mk, ms, slo)
        shi = jnp.where(mk, me, shi)
    seg_ref[0:1, :] = slo
    seg_ref[1:2, :] = shi


def _prep_call(post, brow):
    return pl.pallas_call(
        _prep_body,
        in_specs=[
            pl.BlockSpec((8, NPAD), lambda: (0, 0)),
            pl.BlockSpec((1, NPAD), lambda: (0, 0)),
        ],
        out_specs=[
            pl.BlockSpec((4, NPAD), lambda: (0, 0)),
            pl.BlockSpec((2, NPAD), lambda: (0, 0)),
        ],
        out_shape=[
            jax.ShapeDtypeStruct((4, NPAD), jnp.float32),
            jax.ShapeDtypeStruct((2, NPAD), jnp.int32),
        ],
    )(post, brow)


def _sc_topk_body(p4_hbm, seg_hbm, idx_hbm, p_l, seg_l, out_l):
    wid = lax.axis_index("s") * 2 + lax.axis_index("c")
    base = pl.multiple_of(wid * PWS, PWS)
    pltpu.sync_copy(p4_hbm, p_l)
    pltpu.sync_copy(seg_hbm.at[0, pl.ds(base, PWS)], seg_l.at[0])
    pltpu.sync_copy(seg_hbm.at[1, pl.ds(base, PWS)], seg_l.at[1])

    lane = jnp.arange(16, dtype=jnp.int32)
    inf16 = jnp.full((16,), jnp.inf, jnp.float32)
    mark16 = jnp.full((16,), MARK, jnp.int32)

    def setup(t):
        i = base + t
        t_al = (t // 16) * 16
        tj = jnp.full((16,), t - t_al, jnp.int32)
        sv = seg_l[0, pl.ds(t_al, 16)]
        ev = seg_l[1, pl.ds(t_al, 16)]
        s = lax.reduce_max(jnp.where(lane == tj, sv, -2**31 + 1), axes=(0,))
        e = lax.reduce_max(jnp.where(lane == tj, ev, -2**31 + 1), axes=(0,))
        i_al = (i // 16) * 16
        ij = jnp.full((16,), i - i_al, jnp.int32)
        xi = p_l[0, pl.ds(i_al, 16)].at[ij].get(mode="promise_in_bounds")
        yi = p_l[1, pl.ds(i_al, 16)].at[ij].get(mode="promise_in_bounds")
        zi = p_l[2, pl.ds(i_al, 16)].at[ij].get(mode="promise_in_bounds")
        qi = p_l[3, pl.ds(i_al, 16)].at[ij].get(mode="promise_in_bounds")
        s16 = jnp.full((16,), s, jnp.int32)
        e16 = jnp.full((16,), e, jnp.int32)
        i16 = jnp.full((16,), i, jnp.int32)
        c0 = (s // 16) * 16
        nch = (e - c0 + 15) // 16
        return (xi, yi, zi, qi, s16, e16, i16, c0), nch

    def step(c, st, best):
        xi, yi, zi, qi, s16, e16, i16, c0 = st
        b0k, b0v, b1k, b1v = best
        off = pl.multiple_of(c0 + c * 16, 8)
        xj = p_l[0, pl.ds(off, 16)]
        yj = p_l[1, pl.ds(off, 16)]
        zj = p_l[2, pl.ds(off, 16)]
        qj = p_l[3, pl.ds(off, 16)]
        dot = (xi * xj + yi * yj) + zi * zj
        d2 = (qi + qj) - 2.0 * dot
        colid = jnp.full((16,), off, jnp.int32) + lane
        ok = (colid >= s16) & (colid < e16) & (colid != i16)
        dm = jnp.where(ok, d2, inf16)
        ck, cv = plsc.sort_key_val(dm, colid)
        rk = lax.rev(ck, (0,))
        rv = lax.rev(cv, (0,))
        le = b1k <= rk
        lk = jnp.where(le, b1k, rk)
        lv = jnp.where(le, b1v, rv)
        lk, lv = plsc.sort_key_val(lk, lv)
        rk2 = lax.rev(lk, (0,))
        rv2 = lax.rev(lv, (0,))
        le2 = b0k <= rk2
        n0k = jnp.where(le2, b0k, rk2)
        n0v = jnp.where(le2, b0v, rv2)
        n1k = jnp.where(le2, rk2, b0k)
        n1v = jnp.where(le2, rv2, b0v)
        b0k, b0v = plsc.sort_key_val(n0k, n0v)
        b1k, b1v = plsc.sort_key_val(n1k, n1v)
        return b0k, b0v, b1k, b1v

    def emit(t, best):
        out_l[t // 4, pl.ds((t % 4) * K, 16)] = best[1]
        out_l[t // 4, pl.ds((t % 4) * K + 16, 16)] = best[3]

    def node_pair(j, _):
        t0 = 2 * j
        t1 = 2 * j + 1
        st0, nch0 = setup(t0)
        st1, nch1 = setup(t1)
        init = (inf16, mark16, inf16, mark16)

        def chunk(c, carry):
            a, b = carry
            return step(c, st0, a), step(c, st1, b)

        a, b = lax.fori_loop(0, jnp.maximum(nch0, nch1), chunk, (init, init))
        emit(t0, a)
        emit(t1, b)
        return 0

    lax.fori_loop(0, PWS // 2, node_pair, 0)
    pltpu.sync_copy(out_l,
                    idx_hbm.at[pl.ds(pl.multiple_of(wid * (PWS // 4), 8),
                                     PWS // 4)])


def _sc_topk(p4, seg):
    mesh = plsc.VectorSubcoreMesh(core_axis_name="c", subcore_axis_name="s")
    f = functools.partial(
        pl.kernel,
        mesh=mesh,
        out_type=jax.ShapeDtypeStruct((NSC // 4, 128), jnp.int32),
        scratch_types=[
            pltpu.VMEM((4, NPAD), jnp.float32),
            pltpu.VMEM((2, PWS), jnp.int32),
            pltpu.VMEM((PWS // 4, 128), jnp.int32),
        ],
        compiler_params=pltpu.CompilerParams(use_tc_tiling_on_sc=False,
                                             needs_layout_passes=False),
    )(_sc_topk_body)
    return f(p4, seg).reshape(NSC, K)


def _table_tail(xp, aa, rowid):
    aam = jnp.where(rowid < N, aa, NEG)
    table = jnp.concatenate([xp, aam], axis=1)
    ald = jnp.concatenate([aam[:, H:], jnp.zeros_like(aam[:, H:])], axis=1)
    return table, ald


def _pre_body(z_ref, emb_ref, wc_ref, aa_ref, h_ref, t_ref, ald_ref):
    i = pl.program_id(0)
    rowid = i * BD + lax.broadcasted_iota(jnp.int32, (BD, 1), 0)
    z = z_ref[...]
    oh = (z == lax.broadcasted_iota(jnp.int32, (BD, 104), 1)).astype(jnp.float32)
    h = _dot(oh, emb_ref[...])
    xp = _dot(h, wc_ref[...])
    aa = _dot(xp, aa_ref[...])
    table, ald = _table_tail(xp, aa, rowid)
    h_ref[...] = h
    t_ref[...] = table
    ald_ref[...] = ald


def _pre_call(zcol, embp, wc0, aa0):
    return pl.pallas_call(
        _pre_body,
        grid=(NPAD // BD,),
        in_specs=[
            pl.BlockSpec((BD, 1), lambda i: (i, 0)),
            pl.BlockSpec((104, 128), lambda i: (0, 0)),
            pl.BlockSpec((128, 128), lambda i: (0, 0)),
            pl.BlockSpec((128, 16), lambda i: (0, 0)),
        ],
        out_specs=[
            pl.BlockSpec((BD, 128), lambda i: (i, 0)),
            pl.BlockSpec((BD, TW), lambda i: (i, 0)),
            pl.BlockSpec((BD, 16), lambda i: (i, 0)),
        ],
        out_shape=[
            jax.ShapeDtypeStruct((NPAD, 128), jnp.float32),
            jax.ShapeDtypeStruct((NPAD, TW), jnp.float32),
            jax.ShapeDtypeStruct((NPAD, 16), jnp.float32),
        ],
    )(zcol, embp, wc0, aa0)


def _ffn(o, h, bc, gg, be, wf, bf):
    o = o + bc
    mu = jnp.mean(o, axis=1, keepdims=True)
    xm = o - mu
    v = jnp.mean(xm * xm, axis=1, keepdims=True)
    o = xm * lax.rsqrt(v + 1e-5) * gg + be
    o = _dot(o, wf) + bf
    return 2.0 * h + o


def _mid_body(o_ref, h_ref, bc_ref, g_ref, be_ref, wf_ref, bf_ref,
              wc_ref, aa_ref, hn_ref, t_ref, ald_ref):
    i = pl.program_id(0)
    rowid = i * BD + lax.broadcasted_iota(jnp.int32, (BD, 1), 0)
    hn = _ffn(o_ref[...], h_ref[...], bc_ref[...], g_ref[...], be_ref[...],
              wf_ref[...], bf_ref[...])
    xp = _dot(hn, wc_ref[...])
    aa = _dot(xp, aa_ref[...])
    table, ald = _table_tail(xp, aa, rowid)
    hn_ref[...] = hn
    t_ref[...] = table
    ald_ref[...] = ald


def _mid_call(o, h, bc, gg, be, wf, bf, wc, aa):
    row = lambda i: (i, 0)
    fix = lambda i: (0, 0)
    return pl.pallas_call(
        _mid_body,
        grid=(NPAD // BD,),
        in_specs=[
            pl.BlockSpec((BD, 128), row),
            pl.BlockSpec((BD, 128), row),
            pl.BlockSpec((1, 128), fix),
            pl.BlockSpec((1, 128), fix),
            pl.BlockSpec((1, 128), fix),
            pl.BlockSpec((128, 128), fix),
            pl.BlockSpec((1, 128), fix),
            pl.BlockSpec((128, 128), fix),
            pl.BlockSpec((128, 16), fix),
        ],
        out_specs=[
            pl.BlockSpec((BD, 128), row),
            pl.BlockSpec((BD, TW), row),
            pl.BlockSpec((BD, 16), row),
        ],
        out_shape=[
            jax.ShapeDtypeStruct((NPAD, 128), jnp.float32),
            jax.ShapeDtypeStruct((NPAD, TW), jnp.float32),
            jax.ShapeDtypeStruct((NPAD, 16), jnp.float32),
        ],
    )(o, h, bc, gg, be, wf, bf, wc, aa)


def _post_body(o_ref, h_ref, bc_ref, g_ref, be_ref, wf_ref, bf_ref, hn_ref):
    hn_ref[...] = _ffn(o_ref[...], h_ref[...], bc_ref[...], g_ref[...],
                       be_ref[...], wf_ref[...], bf_ref[...])


def _post_call(o, h, bc, gg, be, wf, bf):
    row = lambda i: (i, 0)
    fix = lambda i: (0, 0)
    return pl.pallas_call(
        _post_body,
        grid=(NPAD // BD,),
        in_specs=[
            pl.BlockSpec((BD, 128), row),
            pl.BlockSpec((BD, 128), row),
            pl.BlockSpec((1, 128), fix),
            pl.BlockSpec((1, 128), fix),
            pl.BlockSpec((1, 128), fix),
            pl.BlockSpec((128, 128), fix),
            pl.BlockSpec((1, 128), fix),
        ],
        out_specs=pl.BlockSpec((BD, 128), row),
        out_shape=jax.ShapeDtypeStruct((NPAD, 128), jnp.float32),
    )(o, h, bc, gg, be, wf, bf)


def _lane(v, j):
    return v.at[jnp.full((16,), j, jnp.int32)].get(mode="promise_in_bounds")


def _sc_gat_body(t_hbm, idx_hbm, ald_hbm, out_hbm,
                 idx_l, ald_l, out_l, escr, rows0, rows1, sem0, sem1):
    wid = lax.axis_index("s") * 2 + lax.axis_index("c")
    base = pl.multiple_of(wid * PW, PW)
    pltpu.sync_copy(idx_hbm.at[pl.ds(pl.multiple_of(wid * (PW // 4), 8),
                                     PW // 4)], idx_l)
    pltpu.sync_copy(ald_hbm.at[pl.ds(pl.multiple_of(wid * (PW // 8), 8),
                                     PW // 8)], ald_l)

    def idx_slice(t):
        return idx_l.at[t // 4, pl.ds((t % 4) * K, K)]

    pltpu.make_async_copy(t_hbm.at[idx_slice(0)], rows0, sem0).start()
    pltpu.make_async_copy(t_hbm.at[idx_slice(1)], rows1, sem1).start()

    def compute(t, rows_ref):
        ald = ald_l[t // 8, pl.ds((t % 8) * 16, 16)]
        m = jnp.full((16,), -3e38, jnp.float32)
        for k in range(K):
            e = rows_ref[k, pl.ds(128, 16)] + ald
            e = jnp.maximum(e, 0.2 * e)
            escr[k // 8, pl.ds((k % 8) * 16, 16)] = e
            m = jnp.maximum(m, e)
        den = jnp.full((16,), 1e-16, jnp.float32)
        num = [jnp.zeros((16,), jnp.float32) for _ in range(H)]
        for k in range(K):
            ex = jnp.exp(escr[k // 8, pl.ds((k % 8) * 16, 16)] - m)
            den = den + ex
            for hi in range(H):
                num[hi] = num[hi] + _lane(ex, hi) * rows_ref[k, pl.ds(hi * C, C)]
        rden = 1.0 / den
        for hi in range(H):
            out_l[t, pl.ds(hi * C, C)] = num[hi] * _lane(rden, hi)

    def body(j, _):
        t0 = 2 * j
        pltpu.make_async_copy(t_hbm.at[idx_slice(t0)], rows0, sem0).wait()
        compute(t0, rows0)

        @pl.when(j < PW // 2 - 1)
        def _():
            pltpu.make_async_copy(t_hbm.at[idx_slice(t0 + 2)], rows0, sem0).start()

        t1 = t0 + 1
        pltpu.make_async_copy(t_hbm.at[idx_slice(t1)], rows1, sem1).wait()
        compute(t1, rows1)

        @pl.when(j < PW // 2 - 1)
        def _():
            pltpu.make_async_copy(t_hbm.at[idx_slice(t1 + 2)], rows1, sem1).start()

        return 0

    lax.fori_loop(0, PW // 2, body, 0)
    pltpu.sync_copy(out_l, out_hbm.at[pl.ds(pl.multiple_of(base, 8), PW)])


def _sc_gat(table, idx, ald):
    mesh = plsc.VectorSubcoreMesh(core_axis_name="c", subcore_axis_name="s")
    f = functools.partial(
        pl.kernel,
        mesh=mesh,
        out_type=jax.ShapeDtypeStruct((NPAD, 128), jnp.float32),
        scratch_types=[
            pltpu.VMEM((PW // 4, 128), jnp.int32),
            pltpu.VMEM((PW // 8, 128), jnp.float32),
            pltpu.VMEM((PW, 128), jnp.float32),
            pltpu.VMEM((K // 8, 128), jnp.float32),
            pltpu.VMEM((K, TW), jnp.float32),
            pltpu.VMEM((K, TW), jnp.float32),
            pltpu.SemaphoreType.DMA,
            pltpu.SemaphoreType.DMA,
        ],
        compiler_params=pltpu.CompilerParams(use_tc_tiling_on_sc=False),
    )(_sc_gat_body)
    return f(table, idx.reshape(NPAD // 4, 128), ald.reshape(NPAD // 8, 128))


def kernel(z, pos, batch, emb, Wc, asrc, adst, bc, g, be, Wf, bf):
    npad = NPAD - N
    posp = jnp.concatenate(
        [pos.astype(jnp.float32), jnp.zeros((npad, 3), jnp.float32)], axis=0)
    posp = jnp.concatenate([posp, jnp.zeros((NPAD, 5), jnp.float32)], axis=1)
    post = posp[:, :8].T
    bpad = jnp.concatenate(
        [batch.astype(jnp.int32), jnp.full((npad,), 16, jnp.int32)])
    brow = bpad.reshape(1, NPAD)
    bcol = bpad.reshape(NPAD, 1)
    zcol = jnp.concatenate(
        [z.astype(jnp.int32), jnp.zeros((npad,), jnp.int32)]).reshape(NPAD, 1)
    embp = jnp.concatenate([emb, jnp.zeros((4, D), jnp.float32)], axis=0)

    rows = jnp.arange(H * C)
    hd = rows // C
    sel = (hd[:, None] == jnp.arange(H)[None, :]).astype(jnp.float32)
    aas = [jnp.concatenate([sel * asrc[b].reshape(-1)[:, None],
                            sel * adst[b].reshape(-1)[:, None]], axis=1)
           for b in range(NB)]

    p4, seg = _prep_call(post, brow)
    idx_sc = _sc_topk(p4, seg)
    idx_tc = _nbr_call(posp, post, brow, bcol)
    idx = jnp.concatenate([idx_sc, idx_tc], axis=0)

    h, table, ald = _pre_call(zcol, embp, Wc[0], aas[0])
    for b in range(NB):
        o = _sc_gat(table, idx, ald)
        if b < NB - 1:
            h, table, ald = _mid_call(
                o, h, bc[b].reshape(1, -1), g[b].reshape(1, -1),
                be[b].reshape(1, -1), Wf[b], bf[b].reshape(1, -1),
                Wc[b + 1], aas[b + 1])
        else:
            h = _post_call(
                o, h, bc[b].reshape(1, -1), g[b].reshape(1, -1),
                be[b].reshape(1, -1), Wf[b], bf[b].reshape(1, -1))
    return h[:N]


# speedup vs baseline: 1.7011x; 1.0010x over previous
import functools

import jax
import jax.numpy as jnp
from jax import lax
from jax.experimental import pallas as pl
from jax.experimental.pallas import tpu as pltpu
from jax.experimental.pallas import tpu_sc as plsc

N = 10000
D = 128
H = 8
C = 16
K = 32
CUT = 5.0
NB = 3

NPAD = 10240
MARK = 10000
RB = 256
CB = 256
BD = 512
NW = 32
PW = NPAD // NW
TW = 144
NEG = -3.0e8

_HI = lax.Precision.HIGHEST


def _dot(a, b):
    return lax.dot_general(a, b, (((1,), (0,)), ((), ())), precision=_HI,
                           preferred_element_type=jnp.float32)


def _nbr_body(posr_ref, post_ref, brow_ref, bcol_ref, idx_ref):
    i = pl.program_id(0)
    r0 = (i + _OFF) * RB
    posr = posr_ref[...]
    sqr = jnp.sum(posr * posr, axis=1, keepdims=True)
    brow = brow_ref[...]
    batch_r = bcol_ref[...]
    rowid = r0 + lax.broadcasted_iota(jnp.int32, (RB, 1), 0)

    bmin = jnp.min(batch_r)
    bmax = jnp.max(batch_r)
    col_iota = lax.broadcasted_iota(jnp.int32, (1, NPAD), 1)
    s = jnp.min(jnp.where(brow == bmin, col_iota, NPAD))
    e = jnp.max(jnp.where(brow == bmax, col_iota, -1))
    c0 = s // CB
    c1 = e // CB

    cpos = lax.broadcasted_iota(jnp.int32, (RB, K + CB), 1)

    def chunk(c, carry):
        bd, bi = carry
        posc = post_ref[:, pl.ds(c * CB, CB)]
        sqc = jnp.sum(posc * posc, axis=0, keepdims=True)
        dot = lax.dot_general(posr, posc, (((1,), (0,)), ((), ())),
                              precision=lax.Precision.DEFAULT,
                              preferred_element_type=jnp.float32)
        d2 = sqr + sqc - 2.0 * dot
        batch_c = brow_ref[:, pl.ds(c * CB, CB)]
        colid = c * CB + lax.broadcasted_iota(jnp.int32, (1, CB), 1)
        valid = ((batch_r == batch_c) & (colid != rowid)
                 & (d2 <= CUT * CUT))
        dm = jnp.where(valid, d2, jnp.inf)
        ci = jnp.where(valid, jnp.broadcast_to(colid, (RB, CB)), MARK)
        cat_d = jnp.concatenate([bd, dm], axis=1)
        cat_i = jnp.concatenate([bi, ci], axis=1)
        nd, ni = [], []
        for _ in range(K):
            m = jnp.min(cat_d, axis=1, keepdims=True)
            am = jnp.min(jnp.where(cat_d == m, cpos, K + CB),
                         axis=1, keepdims=True)
            hit = cpos == am
            vi = jnp.max(jnp.where(hit, cat_i, -1), axis=1, keepdims=True)
            nd.append(m)
            ni.append(vi)
            cat_d = jnp.where(hit, jnp.inf, cat_d)
            cat_i = jnp.where(hit, MARK, cat_i)
        return jnp.concatenate(nd, axis=1), jnp.concatenate(ni, axis=1)

    bd0 = jnp.full((RB, K), jnp.inf, jnp.float32)
    bi0 = jnp.full((RB, K), MARK, jnp.int32)
    _, bi = lax.fori_loop(c0, c1 + 1, chunk, (bd0, bi0))
    idx_ref[...] = bi


NSC = 6144
PWS = NSC // NW
_OFF = NSC // RB


def _nbr_call(posp, post, brow, bcol):
    return pl.pallas_call(
        _nbr_body,
        grid=((NPAD - NSC) // RB,),
        in_specs=[
            pl.BlockSpec((RB, 8), lambda i: (i + _OFF, 0)),
            pl.BlockSpec((8, NPAD), lambda i: (0, 0)),
            pl.BlockSpec((1, NPAD), lambda i: (0, 0)),
            pl.BlockSpec((RB, 1), lambda i: (i + _OFF, 0)),
        ],
        out_specs=pl.BlockSpec((RB, K), lambda i: (i, 0)),
        out_shape=jax.ShapeDtypeStruct((NPAD - NSC, K), jnp.int32),
    )(posp, post, brow, bcol)


def _prep_body(post_ref, brow_ref, p4_ref, seg_ref):
    post = post_ref[...]
    pb = post[0:3].astype(jnp.bfloat16).astype(jnp.float32)
    sq = (post[0] * post[0] + post[1] * post[1]) + post[2] * post[2]
    p4_ref[0:3, :] = pb
    p4_ref[3, :] = sq
    brow = brow_ref[...]
    iota = lax.broadcasted_iota(jnp.int32, (1, NPAD), 1)
    slo = jnp.zeros((1, NPAD), jnp.int32)
    shi = jnp.zeros((1, NPAD), jnp.int32)
    for mmol in range(16):
        mk = brow == mmol
        ms = jnp.min(jnp.where(mk, iota, NPAD))
        me = jnp.max(jnp.where(mk, iota, -1)) + 1
        slo = jnp.where(mk, ms, slo)
        shi = jnp.where(mk, me, shi)
    seg_ref[0:1, :] = slo
    seg_ref[1:2, :] = shi


def _prep_call(post, brow):
    return pl.pallas_call(
        _prep_body,
        in_specs=[
            pl.BlockSpec((8, NPAD), lambda: (0, 0)),
            pl.BlockSpec((1, NPAD), lambda: (0, 0)),
        ],
        out_specs=[
            pl.BlockSpec((4, NPAD), lambda: (0, 0)),
            pl.BlockSpec((2, NPAD), lambda: (0, 0)),
        ],
        out_shape=[
            jax.ShapeDtypeStruct((4, NPAD), jnp.float32),
            jax.ShapeDtypeStruct((2, NPAD), jnp.int32),
        ],
    )(post, brow)


def _sc_topk_body(p4_hbm, seg_hbm, idx_hbm, p_l, seg_l, out_l):
    wid = lax.axis_index("s") * 2 + lax.axis_index("c")
    base = pl.multiple_of(wid * PWS, PWS)
    pltpu.sync_copy(p4_hbm, p_l)
    pltpu.sync_copy(seg_hbm.at[0, pl.ds(base, PWS)], seg_l.at[0])
    pltpu.sync_copy(seg_hbm.at[1, pl.ds(base, PWS)], seg_l.at[1])

    lane = jnp.arange(16, dtype=jnp.int32)
    inf16 = jnp.full((16,), jnp.inf, jnp.float32)
    mark16 = jnp.full((16,), MARK, jnp.int32)

    def setup(t):
        i = base + t
        t_al = (t // 16) * 16
        tj = jnp.full((16,), t - t_al, jnp.int32)
        sv = seg_l[0, pl.ds(t_al, 16)]
        ev = seg_l[1, pl.ds(t_al, 16)]
        s = lax.reduce_max(jnp.where(lane == tj, sv, -2**31 + 1), axes=(0,))
        e = lax.reduce_max(jnp.where(lane == tj, ev, -2**31 + 1), axes=(0,))
        i_al = (i // 16) * 16
        ij = jnp.full((16,), i - i_al, jnp.int32)
        xi = p_l[0, pl.ds(i_al, 16)].at[ij].get(mode="promise_in_bounds")
        yi = p_l[1, pl.ds(i_al, 16)].at[ij].get(mode="promise_in_bounds")
        zi = p_l[2, pl.ds(i_al, 16)].at[ij].get(mode="promise_in_bounds")
        qi = p_l[3, pl.ds(i_al, 16)].at[ij].get(mode="promise_in_bounds")
        s16 = jnp.full((16,), s, jnp.int32)
        e16 = jnp.full((16,), e, jnp.int32)
        i16 = jnp.full((16,), i, jnp.int32)
        c0 = (s // 16) * 16
        nch = (e - c0 + 15) // 16
        return (xi, yi, zi, qi, s16, e16, i16, c0), nch

    def step(c, st, best):
        xi, yi, zi, qi, s16, e16, i16, c0 = st
        b0k, b0v, b1k, b1v = best
        off = pl.multiple_of(c0 + c * 16, 8)
        xj = p_l[0, pl.ds(off, 16)]
        yj = p_l[1, pl.ds(off, 16)]
        zj = p_l[2, pl.ds(off, 16)]
        qj = p_l[3, pl.ds(off, 16)]
        dot = (xi * xj + yi * yj) + zi * zj
        d2 = (qi + qj) - 2.0 * dot
        colid = jnp.full((16,), off, jnp.int32) + lane
        ok = (colid >= s16) & (colid < e16) & (colid != i16)
        dm = jnp.where(ok, d2, inf16)

        def merge(best):
            b0k, b0v, b1k, b1v = best
            ck, cv = plsc.sort_key_val(dm, colid)
            rk = lax.rev(ck, (0,))
            rv = lax.rev(cv, (0,))
            le = b1k <= rk
            lk = jnp.where(le, b1k, rk)
            lv = jnp.where(le, b1v, rv)
            lk, lv = plsc.sort_key_val(lk, lv)
            rk2 = lax.rev(lk, (0,))
            rv2 = lax.rev(lv, (0,))
            le2 = b0k <= rk2
            n0k = jnp.where(le2, b0k, rk2)
            n0v = jnp.where(le2, b0v, rv2)
            n1k = jnp.where(le2, rk2, b0k)
            n1v = jnp.where(le2, rv2, b0v)
            b0k, b0v = plsc.sort_key_val(n0k, n0v)
            b1k, b1v = plsc.sort_key_val(n1k, n1v)
            return b0k, b0v, b1k, b1v

        pred = lax.reduce_min(dm, axes=(0,)) < lax.reduce_max(b1k, axes=(0,))
        return lax.cond(pred, merge, lambda best: best, (b0k, b0v, b1k, b1v))

    def emit(t, best):
        out_l[t // 4, pl.ds((t % 4) * K, 16)] = best[1]
        out_l[t // 4, pl.ds((t % 4) * K + 16, 16)] = best[3]

    def node_pair(j, _):
        t0 = 2 * j
        t1 = 2 * j + 1
        st0, nch0 = setup(t0)
        st1, nch1 = setup(t1)
        init = (inf16, mark16, inf16, mark16)

        def chunk(c, carry):
            a, b = carry
            return step(c, st0, a), step(c, st1, b)

        a, b = lax.fori_loop(0, jnp.maximum(nch0, nch1), chunk, (init, init))
        emit(t0, a)
        emit(t1, b)
        return 0

    lax.fori_loop(0, PWS // 2, node_pair, 0)
    pltpu.sync_copy(out_l,
                    idx_hbm.at[pl.ds(pl.multiple_of(wid * (PWS // 4), 8),
                                     PWS // 4)])


def _sc_topk(p4, seg):
    mesh = plsc.VectorSubcoreMesh(core_axis_name="c", subcore_axis_name="s")
    f = functools.partial(
        pl.kernel,
        mesh=mesh,
        out_type=jax.ShapeDtypeStruct((NSC // 4, 128), jnp.int32),
        scratch_types=[
            pltpu.VMEM((4, NPAD), jnp.float32),
            pltpu.VMEM((2, PWS), jnp.int32),
            pltpu.VMEM((PWS // 4, 128), jnp.int32),
        ],
        compiler_params=pltpu.CompilerParams(use_tc_tiling_on_sc=False,
                                             needs_layout_passes=False),
    )(_sc_topk_body)
    return f(p4, seg).reshape(NSC, K)


def _table_tail(xp, aa, rowid):
    aam = jnp.where(rowid < N, aa, NEG)
    table = jnp.concatenate([xp, aam], axis=1)
    ald = jnp.concatenate([aam[:, H:], jnp.zeros_like(aam[:, H:])], axis=1)
    return table, ald


def _pre_body(z_ref, emb_ref, wc_ref, aa_ref, h_ref, t_ref, ald_ref):
    i = pl.program_id(0)
    rowid = i * BD + lax.broadcasted_iota(jnp.int32, (BD, 1), 0)
    z = z_ref[...]
    oh = (z == lax.broadcasted_iota(jnp.int32, (BD, 104), 1)).astype(jnp.float32)
    h = _dot(oh, emb_ref[...])
    xp = _dot(h, wc_ref[...])
    aa = _dot(xp, aa_ref[...])
    table, ald = _table_tail(xp, aa, rowid)
    h_ref[...] = h
    t_ref[...] = table
    ald_ref[...] = ald


def _pre_call(zcol, embp, wc0, aa0):
    return pl.pallas_call(
        _pre_body,
        grid=(NPAD // BD,),
        in_specs=[
            pl.BlockSpec((BD, 1), lambda i: (i, 0)),
            pl.BlockSpec((104, 128), lambda i: (0, 0)),
            pl.BlockSpec((128, 128), lambda i: (0, 0)),
            pl.BlockSpec((128, 16), lambda i: (0, 0)),
        ],
        out_specs=[
            pl.BlockSpec((BD, 128), lambda i: (i, 0)),
            pl.BlockSpec((BD, TW), lambda i: (i, 0)),
            pl.BlockSpec((BD, 16), lambda i: (i, 0)),
        ],
        out_shape=[
            jax.ShapeDtypeStruct((NPAD, 128), jnp.float32),
            jax.ShapeDtypeStruct((NPAD, TW), jnp.float32),
            jax.ShapeDtypeStruct((NPAD, 16), jnp.float32),
        ],
    )(zcol, embp, wc0, aa0)


def _ffn(o, h, bc, gg, be, wf, bf):
    o = o + bc
    mu = jnp.mean(o, axis=1, keepdims=True)
    xm = o - mu
    v = jnp.mean(xm * xm, axis=1, keepdims=True)
    o = xm * lax.rsqrt(v + 1e-5) * gg + be
    o = _dot(o, wf) + bf
    return 2.0 * h + o


def _mid_body(o_ref, h_ref, bc_ref, g_ref, be_ref, wf_ref, bf_ref,
              wc_ref, aa_ref, hn_ref, t_ref, ald_ref):
    i = pl.program_id(0)
    rowid = i * BD + lax.broadcasted_iota(jnp.int32, (BD, 1), 0)
    hn = _ffn(o_ref[...], h_ref[...], bc_ref[...], g_ref[...], be_ref[...],
              wf_ref[...], bf_ref[...])
    xp = _dot(hn, wc_ref[...])
    aa = _dot(xp, aa_ref[...])
    table, ald = _table_tail(xp, aa, rowid)
    hn_ref[...] = hn
    t_ref[...] = table
    ald_ref[...] = ald


def _mid_call(o, h, bc, gg, be, wf, bf, wc, aa):
    row = lambda i: (i, 0)
    fix = lambda i: (0, 0)
    return pl.pallas_call(
        _mid_body,
        grid=(NPAD // BD,),
        in_specs=[
            pl.BlockSpec((BD, 128), row),
            pl.BlockSpec((BD, 128), row),
            pl.BlockSpec((1, 128), fix),
            pl.BlockSpec((1, 128), fix),
            pl.BlockSpec((1, 128), fix),
            pl.BlockSpec((128, 128), fix),
            pl.BlockSpec((1, 128), fix),
            pl.BlockSpec((128, 128), fix),
            pl.BlockSpec((128, 16), fix),
        ],
        out_specs=[
            pl.BlockSpec((BD, 128), row),
            pl.BlockSpec((BD, TW), row),
            pl.BlockSpec((BD, 16), row),
        ],
        out_shape=[
            jax.ShapeDtypeStruct((NPAD, 128), jnp.float32),
            jax.ShapeDtypeStruct((NPAD, TW), jnp.float32),
            jax.ShapeDtypeStruct((NPAD, 16), jnp.float32),
        ],
    )(o, h, bc, gg, be, wf, bf, wc, aa)


def _post_body(o_ref, h_ref, bc_ref, g_ref, be_ref, wf_ref, bf_ref, hn_ref):
    hn_ref[...] = _ffn(o_ref[...], h_ref[...], bc_ref[...], g_ref[...],
                       be_ref[...], wf_ref[...], bf_ref[...])


def _post_call(o, h, bc, gg, be, wf, bf):
    row = lambda i: (i, 0)
    fix = lambda i: (0, 0)
    return pl.pallas_call(
        _post_body,
        grid=(NPAD // BD,),
        in_specs=[
            pl.BlockSpec((BD, 128), row),
            pl.BlockSpec((BD, 128), row),
            pl.BlockSpec((1, 128), fix),
            pl.BlockSpec((1, 128), fix),
            pl.BlockSpec((1, 128), fix),
            pl.BlockSpec((128, 128), fix),
            pl.BlockSpec((1, 128), fix),
        ],
        out_specs=pl.BlockSpec((BD, 128), row),
        out_shape=jax.ShapeDtypeStruct((NPAD, 128), jnp.float32),
    )(o, h, bc, gg, be, wf, bf)


def _lane(v, j):
    return v.at[jnp.full((16,), j, jnp.int32)].get(mode="promise_in_bounds")


def _sc_gat_body(t_hbm, idx_hbm, ald_hbm, out_hbm,
                 idx_l, ald_l, out_l, escr, rows0, rows1, sem0, sem1):
    wid = lax.axis_index("s") * 2 + lax.axis_index("c")
    base = pl.multiple_of(wid * PW, PW)
    pltpu.sync_copy(idx_hbm.at[pl.ds(pl.multiple_of(wid * (PW // 4), 8),
                                     PW // 4)], idx_l)
    pltpu.sync_copy(ald_hbm.at[pl.ds(pl.multiple_of(wid * (PW // 8), 8),
                                     PW // 8)], ald_l)

    def idx_slice(t):
        return idx_l.at[t // 4, pl.ds((t % 4) * K, K)]

    pltpu.make_async_copy(t_hbm.at[idx_slice(0)], rows0, sem0).start()
    pltpu.make_async_copy(t_hbm.at[idx_slice(1)], rows1, sem1).start()

    def compute(t, rows_ref):
        ald = ald_l[t // 8, pl.ds((t % 8) * 16, 16)]
        m = jnp.full((16,), -3e38, jnp.float32)
        for k in range(K):
            e = rows_ref[k, pl.ds(128, 16)] + ald
            e = jnp.maximum(e, 0.2 * e)
            escr[k // 8, pl.ds((k % 8) * 16, 16)] = e
            m = jnp.maximum(m, e)
        den = jnp.full((16,), 1e-16, jnp.float32)
        num = [jnp.zeros((16,), jnp.float32) for _ in range(H)]
        for k in range(K):
            ex = jnp.exp(escr[k // 8, pl.ds((k % 8) * 16, 16)] - m)
            den = den + ex
            for hi in range(H):
                num[hi] = num[hi] + _lane(ex, hi) * rows_ref[k, pl.ds(hi * C, C)]
        rden = 1.0 / den
        for hi in range(H):
            out_l[t, pl.ds(hi * C, C)] = num[hi] * _lane(rden, hi)

    def body(j, _):
        t0 = 2 * j
        pltpu.make_async_copy(t_hbm.at[idx_slice(t0)], rows0, sem0).wait()
        compute(t0, rows0)

        @pl.when(j < PW // 2 - 1)
        def _():
            pltpu.make_async_copy(t_hbm.at[idx_slice(t0 + 2)], rows0, sem0).start()

        t1 = t0 + 1
        pltpu.make_async_copy(t_hbm.at[idx_slice(t1)], rows1, sem1).wait()
        compute(t1, rows1)

        @pl.when(j < PW // 2 - 1)
        def _():
            pltpu.make_async_copy(t_hbm.at[idx_slice(t1 + 2)], rows1, sem1).start()

        return 0

    lax.fori_loop(0, PW // 2, body, 0)
    pltpu.sync_copy(out_l, out_hbm.at[pl.ds(pl.multiple_of(base, 8), PW)])


def _sc_gat(table, idx, ald):
    mesh = plsc.VectorSubcoreMesh(core_axis_name="c", subcore_axis_name="s")
    f = functools.partial(
        pl.kernel,
        mesh=mesh,
        out_type=jax.ShapeDtypeStruct((NPAD, 128), jnp.float32),
        scratch_types=[
            pltpu.VMEM((PW // 4, 128), jnp.int32),
            pltpu.VMEM((PW // 8, 128), jnp.float32),
            pltpu.VMEM((PW, 128), jnp.float32),
            pltpu.VMEM((K // 8, 128), jnp.float32),
            pltpu.VMEM((K, TW), jnp.float32),
            pltpu.VMEM((K, TW), jnp.float32),
            pltpu.SemaphoreType.DMA,
            pltpu.SemaphoreType.DMA,
        ],
        compiler_params=pltpu.CompilerParams(use_tc_tiling_on_sc=False),
    )(_sc_gat_body)
    return f(table, idx.reshape(NPAD // 4, 128), ald.reshape(NPAD // 8, 128))


def kernel(z, pos, batch, emb, Wc, asrc, adst, bc, g, be, Wf, bf):
    npad = NPAD - N
    posp = jnp.concatenate(
        [pos.astype(jnp.float32), jnp.zeros((npad, 3), jnp.float32)], axis=0)
    posp = jnp.concatenate([posp, jnp.zeros((NPAD, 5), jnp.float32)], axis=1)
    post = posp[:, :8].T
    bpad = jnp.concatenate(
        [batch.astype(jnp.int32), jnp.full((npad,), 16, jnp.int32)])
    brow = bpad.reshape(1, NPAD)
    bcol = bpad.reshape(NPAD, 1)
    zcol = jnp.concatenate(
        [z.astype(jnp.int32), jnp.zeros((npad,), jnp.int32)]).reshape(NPAD, 1)
    embp = jnp.concatenate([emb, jnp.zeros((4, D), jnp.float32)], axis=0)

    rows = jnp.arange(H * C)
    hd = rows // C
    sel = (hd[:, None] == jnp.arange(H)[None, :]).astype(jnp.float32)
    aas = [jnp.concatenate([sel * asrc[b].reshape(-1)[:, None],
                            sel * adst[b].reshape(-1)[:, None]], axis=1)
           for b in range(NB)]

    p4, seg = _prep_call(post, brow)
    idx_sc = _sc_topk(p4, seg)
    idx_tc = _nbr_call(posp, post, brow, bcol)
    idx = jnp.concatenate([idx_sc, idx_tc], axis=0)

    h, table, ald = _pre_call(zcol, embp, Wc[0], aas[0])
    for b in range(NB):
        o = _sc_gat(table, idx, ald)
        if b < NB - 1:
            h, table, ald = _mid_call(
                o, h, bc[b].reshape(1, -1), g[b].reshape(1, -1),
                be[b].reshape(1, -1), Wf[b], bf[b].reshape(1, -1),
                Wc[b + 1], aas[b + 1])
        else:
            h = _post_call(
                o, h, bc[b].reshape(1, -1), g[b].reshape(1, -1),
                be[b].reshape(1, -1), Wf[b], bf[b].reshape(1, -1))
    return h[:N]


# speedup vs baseline: 1.9375x; 1.1390x over previous
import functools

import jax
import jax.numpy as jnp
from jax import lax
from jax.experimental import pallas as pl
from jax.experimental.pallas import tpu as pltpu
from jax.experimental.pallas import tpu_sc as plsc

N = 10000
D = 128
H = 8
C = 16
K = 32
CUT = 5.0
NB = 3

NPAD = 10240
MARK = 10000
RB = 256
CB = 256
BD = 512
NW = 32
PW = NPAD // NW
TW = 144
NEG = -3.0e8

_HI = lax.Precision.HIGHEST


def _dot(a, b):
    return lax.dot_general(a, b, (((1,), (0,)), ((), ())), precision=_HI,
                           preferred_element_type=jnp.float32)


def _nbr_body(posr_ref, post_ref, brow_ref, bcol_ref, idx_ref):
    i = pl.program_id(0)
    r0 = (i + _OFF) * RB
    posr = posr_ref[...]
    sqr = jnp.sum(posr * posr, axis=1, keepdims=True)
    brow = brow_ref[...]
    batch_r = bcol_ref[...]
    rowid = r0 + lax.broadcasted_iota(jnp.int32, (RB, 1), 0)

    bmin = jnp.min(batch_r)
    bmax = jnp.max(batch_r)
    col_iota = lax.broadcasted_iota(jnp.int32, (1, NPAD), 1)
    s = jnp.min(jnp.where(brow == bmin, col_iota, NPAD))
    e = jnp.max(jnp.where(brow == bmax, col_iota, -1))
    c0 = s // CB
    c1 = e // CB

    cpos = lax.broadcasted_iota(jnp.int32, (RB, K + CB), 1)

    def chunk(c, carry):
        bd, bi = carry
        posc = post_ref[:, pl.ds(c * CB, CB)]
        sqc = jnp.sum(posc * posc, axis=0, keepdims=True)
        dot = lax.dot_general(posr, posc, (((1,), (0,)), ((), ())),
                              precision=lax.Precision.DEFAULT,
                              preferred_element_type=jnp.float32)
        d2 = sqr + sqc - 2.0 * dot
        batch_c = brow_ref[:, pl.ds(c * CB, CB)]
        colid = c * CB + lax.broadcasted_iota(jnp.int32, (1, CB), 1)
        valid = ((batch_r == batch_c) & (colid != rowid)
                 & (d2 <= CUT * CUT))
        dm = jnp.where(valid, d2, jnp.inf)
        ci = jnp.where(valid, jnp.broadcast_to(colid, (RB, CB)), MARK)
        cat_d = jnp.concatenate([bd, dm], axis=1)
        cat_i = jnp.concatenate([bi, ci], axis=1)
        nd, ni = [], []
        for _ in range(K):
            m = jnp.min(cat_d, axis=1, keepdims=True)
            am = jnp.min(jnp.where(cat_d == m, cpos, K + CB),
                         axis=1, keepdims=True)
            hit = cpos == am
            vi = jnp.max(jnp.where(hit, cat_i, -1), axis=1, keepdims=True)
            nd.append(m)
            ni.append(vi)
            cat_d = jnp.where(hit, jnp.inf, cat_d)
            cat_i = jnp.where(hit, MARK, cat_i)
        return jnp.concatenate(nd, axis=1), jnp.concatenate(ni, axis=1)

    bd0 = jnp.full((RB, K), jnp.inf, jnp.float32)
    bi0 = jnp.full((RB, K), MARK, jnp.int32)
    _, bi = lax.fori_loop(c0, c1 + 1, chunk, (bd0, bi0))
    idx_ref[...] = bi


NSC = 7168
PWS = NSC // NW
_OFF = NSC // RB


def _nbr_call(posp, post, brow, bcol):
    return pl.pallas_call(
        _nbr_body,
        grid=((NPAD - NSC) // RB,),
        in_specs=[
            pl.BlockSpec((RB, 8), lambda i: (i + _OFF, 0)),
            pl.BlockSpec((8, NPAD), lambda i: (0, 0)),
            pl.BlockSpec((1, NPAD), lambda i: (0, 0)),
            pl.BlockSpec((RB, 1), lambda i: (i + _OFF, 0)),
        ],
        out_specs=pl.BlockSpec((RB, K), lambda i: (i, 0)),
        out_shape=jax.ShapeDtypeStruct((NPAD - NSC, K), jnp.int32),
    )(posp, post, brow, bcol)


def _prep_body(post_ref, brow_ref, p4_ref, seg_ref):
    post = post_ref[...]
    pb = post[0:3].astype(jnp.bfloat16).astype(jnp.float32)
    sq = (post[0] * post[0] + post[1] * post[1]) + post[2] * post[2]
    p4_ref[0:3, :] = pb
    p4_ref[3, :] = sq
    brow = brow_ref[...]
    iota = lax.broadcasted_iota(jnp.int32, (1, NPAD), 1)
    slo = jnp.zeros((1, NPAD), jnp.int32)
    shi = jnp.zeros((1, NPAD), jnp.int32)
    for mmol in range(16):
        mk = brow == mmol
        ms = jnp.min(jnp.where(mk, iota, NPAD))
        me = jnp.max(jnp.where(mk, iota, -1)) + 1
        slo = jnp.where(mk, ms, slo)
        shi = jnp.where(mk, me, shi)
    seg_ref[0:1, :] = slo
    seg_ref[1:2, :] = shi


def _prep_call(post, brow):
    return pl.pallas_call(
        _prep_body,
        in_specs=[
            pl.BlockSpec((8, NPAD), lambda: (0, 0)),
            pl.BlockSpec((1, NPAD), lambda: (0, 0)),
        ],
        out_specs=[
            pl.BlockSpec((4, NPAD), lambda: (0, 0)),
            pl.BlockSpec((2, NPAD), lambda: (0, 0)),
        ],
        out_shape=[
            jax.ShapeDtypeStruct((4, NPAD), jnp.float32),
            jax.ShapeDtypeStruct((2, NPAD), jnp.int32),
        ],
    )(post, brow)


def _sc_topk_body(p4_hbm, seg_hbm, idx_hbm, p_l, seg_l, out_l):
    wid = lax.axis_index("s") * 2 + lax.axis_index("c")
    base = pl.multiple_of(wid * PWS, PWS)
    pltpu.sync_copy(p4_hbm, p_l)
    pltpu.sync_copy(seg_hbm.at[0, pl.ds(base, PWS)], seg_l.at[0])
    pltpu.sync_copy(seg_hbm.at[1, pl.ds(base, PWS)], seg_l.at[1])

    lane = jnp.arange(16, dtype=jnp.int32)
    inf16 = jnp.full((16,), jnp.inf, jnp.float32)
    mark16 = jnp.full((16,), MARK, jnp.int32)

    def setup(t):
        i = base + t
        t_al = (t // 16) * 16
        tj = jnp.full((16,), t - t_al, jnp.int32)
        sv = seg_l[0, pl.ds(t_al, 16)]
        ev = seg_l[1, pl.ds(t_al, 16)]
        s = lax.reduce_max(jnp.where(lane == tj, sv, -2**31 + 1), axes=(0,))
        e = lax.reduce_max(jnp.where(lane == tj, ev, -2**31 + 1), axes=(0,))
        i_al = (i // 16) * 16
        ij = jnp.full((16,), i - i_al, jnp.int32)
        xi = p_l[0, pl.ds(i_al, 16)].at[ij].get(mode="promise_in_bounds")
        yi = p_l[1, pl.ds(i_al, 16)].at[ij].get(mode="promise_in_bounds")
        zi = p_l[2, pl.ds(i_al, 16)].at[ij].get(mode="promise_in_bounds")
        qi = p_l[3, pl.ds(i_al, 16)].at[ij].get(mode="promise_in_bounds")
        s16 = jnp.full((16,), s, jnp.int32)
        e16 = jnp.full((16,), e, jnp.int32)
        i16 = jnp.full((16,), i, jnp.int32)
        c0 = (s // 16) * 16
        nch = (e - c0 + 15) // 16
        return (xi, yi, zi, qi, s16, e16, i16, c0), nch

    def step(c, st, best):
        xi, yi, zi, qi, s16, e16, i16, c0 = st
        b0k, b0v, b1k, b1v = best
        off = pl.multiple_of(c0 + c * 16, 8)
        xj = p_l[0, pl.ds(off, 16)]
        yj = p_l[1, pl.ds(off, 16)]
        zj = p_l[2, pl.ds(off, 16)]
        qj = p_l[3, pl.ds(off, 16)]
        dot = (xi * xj + yi * yj) + zi * zj
        d2 = (qi + qj) - 2.0 * dot
        colid = jnp.full((16,), off, jnp.int32) + lane
        ok = (colid >= s16) & (colid < e16) & (colid != i16)
        dm = jnp.where(ok, d2, inf16)

        def merge(best):
            b0k, b0v, b1k, b1v = best
            ck, cv = plsc.sort_key_val(dm, colid)
            rk = lax.rev(ck, (0,))
            rv = lax.rev(cv, (0,))
            le = b1k <= rk
            lk = jnp.where(le, b1k, rk)
            lv = jnp.where(le, b1v, rv)
            lk, lv = plsc.sort_key_val(lk, lv)
            rk2 = lax.rev(lk, (0,))
            rv2 = lax.rev(lv, (0,))
            le2 = b0k <= rk2
            n0k = jnp.where(le2, b0k, rk2)
            n0v = jnp.where(le2, b0v, rv2)
            n1k = jnp.where(le2, rk2, b0k)
            n1v = jnp.where(le2, rv2, b0v)
            b0k, b0v = plsc.sort_key_val(n0k, n0v)
            b1k, b1v = plsc.sort_key_val(n1k, n1v)
            return b0k, b0v, b1k, b1v

        pred = lax.reduce_min(dm, axes=(0,)) < lax.reduce_max(b1k, axes=(0,))
        return lax.cond(pred, merge, lambda best: best, (b0k, b0v, b1k, b1v))

    def emit(t, best):
        out_l[t // 4, pl.ds((t % 4) * K, 16)] = best[1]
        out_l[t // 4, pl.ds((t % 4) * K + 16, 16)] = best[3]

    def node_pair(j, _):
        t0 = 2 * j
        t1 = 2 * j + 1
        st0, nch0 = setup(t0)
        st1, nch1 = setup(t1)
        init = (inf16, mark16, inf16, mark16)

        def chunk(c, carry):
            a, b = carry
            return step(c, st0, a), step(c, st1, b)

        a, b = lax.fori_loop(0, jnp.maximum(nch0, nch1), chunk, (init, init))
        emit(t0, a)
        emit(t1, b)
        return 0

    lax.fori_loop(0, PWS // 2, node_pair, 0)
    pltpu.sync_copy(out_l,
                    idx_hbm.at[pl.ds(pl.multiple_of(wid * (PWS // 4), 8),
                                     PWS // 4)])


def _sc_topk(p4, seg):
    mesh = plsc.VectorSubcoreMesh(core_axis_name="c", subcore_axis_name="s")
    f = functools.partial(
        pl.kernel,
        mesh=mesh,
        out_type=jax.ShapeDtypeStruct((NSC // 4, 128), jnp.int32),
        scratch_types=[
            pltpu.VMEM((4, NPAD), jnp.float32),
            pltpu.VMEM((2, PWS), jnp.int32),
            pltpu.VMEM((PWS // 4, 128), jnp.int32),
        ],
        compiler_params=pltpu.CompilerParams(use_tc_tiling_on_sc=False,
                                             needs_layout_passes=False),
    )(_sc_topk_body)
    return f(p4, seg).reshape(NSC, K)


def _table_tail(xp, aa, rowid):
    aam = jnp.where(rowid < N, aa, NEG)
    table = jnp.concatenate([xp, aam], axis=1)
    ald = jnp.concatenate([aam[:, H:], jnp.zeros_like(aam[:, H:])], axis=1)
    return table, ald


def _pre_body(z_ref, emb_ref, wc_ref, aa_ref, h_ref, t_ref, ald_ref):
    i = pl.program_id(0)
    rowid = i * BD + lax.broadcasted_iota(jnp.int32, (BD, 1), 0)
    z = z_ref[...]
    oh = (z == lax.broadcasted_iota(jnp.int32, (BD, 104), 1)).astype(jnp.float32)
    h = _dot(oh, emb_ref[...])
    xp = _dot(h, wc_ref[...])
    aa = _dot(xp, aa_ref[...])
    table, ald = _table_tail(xp, aa, rowid)
    h_ref[...] = h
    t_ref[...] = table
    ald_ref[...] = ald


def _pre_call(zcol, embp, wc0, aa0):
    return pl.pallas_call(
        _pre_body,
        grid=(NPAD // BD,),
        in_specs=[
            pl.BlockSpec((BD, 1), lambda i: (i, 0)),
            pl.BlockSpec((104, 128), lambda i: (0, 0)),
            pl.BlockSpec((128, 128), lambda i: (0, 0)),
            pl.BlockSpec((128, 16), lambda i: (0, 0)),
        ],
        out_specs=[
            pl.BlockSpec((BD, 128), lambda i: (i, 0)),
            pl.BlockSpec((BD, TW), lambda i: (i, 0)),
            pl.BlockSpec((BD, 16), lambda i: (i, 0)),
        ],
        out_shape=[
            jax.ShapeDtypeStruct((NPAD, 128), jnp.float32),
            jax.ShapeDtypeStruct((NPAD, TW), jnp.float32),
            jax.ShapeDtypeStruct((NPAD, 16), jnp.float32),
        ],
    )(zcol, embp, wc0, aa0)


def _ffn(o, h, bc, gg, be, wf, bf):
    o = o + bc
    mu = jnp.mean(o, axis=1, keepdims=True)
    xm = o - mu
    v = jnp.mean(xm * xm, axis=1, keepdims=True)
    o = xm * lax.rsqrt(v + 1e-5) * gg + be
    o = _dot(o, wf) + bf
    return 2.0 * h + o


def _mid_body(o_ref, h_ref, bc_ref, g_ref, be_ref, wf_ref, bf_ref,
              wc_ref, aa_ref, hn_ref, t_ref, ald_ref):
    i = pl.program_id(0)
    rowid = i * BD + lax.broadcasted_iota(jnp.int32, (BD, 1), 0)
    hn = _ffn(o_ref[...], h_ref[...], bc_ref[...], g_ref[...], be_ref[...],
              wf_ref[...], bf_ref[...])
    xp = _dot(hn, wc_ref[...])
    aa = _dot(xp, aa_ref[...])
    table, ald = _table_tail(xp, aa, rowid)
    hn_ref[...] = hn
    t_ref[...] = table
    ald_ref[...] = ald


def _mid_call(o, h, bc, gg, be, wf, bf, wc, aa):
    row = lambda i: (i, 0)
    fix = lambda i: (0, 0)
    return pl.pallas_call(
        _mid_body,
        grid=(NPAD // BD,),
        in_specs=[
            pl.BlockSpec((BD, 128), row),
            pl.BlockSpec((BD, 128), row),
            pl.BlockSpec((1, 128), fix),
            pl.BlockSpec((1, 128), fix),
            pl.BlockSpec((1, 128), fix),
            pl.BlockSpec((128, 128), fix),
            pl.BlockSpec((1, 128), fix),
            pl.BlockSpec((128, 128), fix),
            pl.BlockSpec((128, 16), fix),
        ],
        out_specs=[
            pl.BlockSpec((BD, 128), row),
            pl.BlockSpec((BD, TW), row),
            pl.BlockSpec((BD, 16), row),
        ],
        out_shape=[
            jax.ShapeDtypeStruct((NPAD, 128), jnp.float32),
            jax.ShapeDtypeStruct((NPAD, TW), jnp.float32),
            jax.ShapeDtypeStruct((NPAD, 16), jnp.float32),
        ],
    )(o, h, bc, gg, be, wf, bf, wc, aa)


def _post_body(o_ref, h_ref, bc_ref, g_ref, be_ref, wf_ref, bf_ref, hn_ref):
    hn_ref[...] = _ffn(o_ref[...], h_ref[...], bc_ref[...], g_ref[...],
                       be_ref[...], wf_ref[...], bf_ref[...])


def _post_call(o, h, bc, gg, be, wf, bf):
    row = lambda i: (i, 0)
    fix = lambda i: (0, 0)
    return pl.pallas_call(
        _post_body,
        grid=(NPAD // BD,),
        in_specs=[
            pl.BlockSpec((BD, 128), row),
            pl.BlockSpec((BD, 128), row),
            pl.BlockSpec((1, 128), fix),
            pl.BlockSpec((1, 128), fix),
            pl.BlockSpec((1, 128), fix),
            pl.BlockSpec((128, 128), fix),
            pl.BlockSpec((1, 128), fix),
        ],
        out_specs=pl.BlockSpec((BD, 128), row),
        out_shape=jax.ShapeDtypeStruct((NPAD, 128), jnp.float32),
    )(o, h, bc, gg, be, wf, bf)


def _lane(v, j):
    return v.at[jnp.full((16,), j, jnp.int32)].get(mode="promise_in_bounds")


def _sc_gat_body(t_hbm, idx_hbm, ald_hbm, out_hbm,
                 idx_l, ald_l, out_l, escr, rows0, rows1, sem0, sem1):
    wid = lax.axis_index("s") * 2 + lax.axis_index("c")
    base = pl.multiple_of(wid * PW, PW)
    pltpu.sync_copy(idx_hbm.at[pl.ds(pl.multiple_of(wid * (PW // 4), 8),
                                     PW // 4)], idx_l)
    pltpu.sync_copy(ald_hbm.at[pl.ds(pl.multiple_of(wid * (PW // 8), 8),
                                     PW // 8)], ald_l)

    def idx_slice(t):
        return idx_l.at[t // 4, pl.ds((t % 4) * K, K)]

    pltpu.make_async_copy(t_hbm.at[idx_slice(0)], rows0, sem0).start()
    pltpu.make_async_copy(t_hbm.at[idx_slice(1)], rows1, sem1).start()

    def compute(t, rows_ref):
        ald = ald_l[t // 8, pl.ds((t % 8) * 16, 16)]
        m = jnp.full((16,), -3e38, jnp.float32)
        for k in range(K):
            e = rows_ref[k, pl.ds(128, 16)] + ald
            e = jnp.maximum(e, 0.2 * e)
            escr[k // 8, pl.ds((k % 8) * 16, 16)] = e
            m = jnp.maximum(m, e)
        den = jnp.full((16,), 1e-16, jnp.float32)
        num = [jnp.zeros((16,), jnp.float32) for _ in range(H)]
        for k in range(K):
            ex = jnp.exp(escr[k // 8, pl.ds((k % 8) * 16, 16)] - m)
            den = den + ex
            for hi in range(H):
                num[hi] = num[hi] + _lane(ex, hi) * rows_ref[k, pl.ds(hi * C, C)]
        rden = 1.0 / den
        for hi in range(H):
            out_l[t, pl.ds(hi * C, C)] = num[hi] * _lane(rden, hi)

    def body(j, _):
        t0 = 2 * j
        pltpu.make_async_copy(t_hbm.at[idx_slice(t0)], rows0, sem0).wait()
        compute(t0, rows0)

        @pl.when(j < PW // 2 - 1)
        def _():
            pltpu.make_async_copy(t_hbm.at[idx_slice(t0 + 2)], rows0, sem0).start()

        t1 = t0 + 1
        pltpu.make_async_copy(t_hbm.at[idx_slice(t1)], rows1, sem1).wait()
        compute(t1, rows1)

        @pl.when(j < PW // 2 - 1)
        def _():
            pltpu.make_async_copy(t_hbm.at[idx_slice(t1 + 2)], rows1, sem1).start()

        return 0

    lax.fori_loop(0, PW // 2, body, 0)
    pltpu.sync_copy(out_l, out_hbm.at[pl.ds(pl.multiple_of(base, 8), PW)])


def _sc_gat(table, idx, ald):
    mesh = plsc.VectorSubcoreMesh(core_axis_name="c", subcore_axis_name="s")
    f = functools.partial(
        pl.kernel,
        mesh=mesh,
        out_type=jax.ShapeDtypeStruct((NPAD, 128), jnp.float32),
        scratch_types=[
            pltpu.VMEM((PW // 4, 128), jnp.int32),
            pltpu.VMEM((PW // 8, 128), jnp.float32),
            pltpu.VMEM((PW, 128), jnp.float32),
            pltpu.VMEM((K // 8, 128), jnp.float32),
            pltpu.VMEM((K, TW), jnp.float32),
            pltpu.VMEM((K, TW), jnp.float32),
            pltpu.SemaphoreType.DMA,
            pltpu.SemaphoreType.DMA,
        ],
        compiler_params=pltpu.CompilerParams(use_tc_tiling_on_sc=False),
    )(_sc_gat_body)
    return f(table, idx.reshape(NPAD // 4, 128), ald.reshape(NPAD // 8, 128))


def kernel(z, pos, batch, emb, Wc, asrc, adst, bc, g, be, Wf, bf):
    npad = NPAD - N
    posp = jnp.concatenate(
        [pos.astype(jnp.float32), jnp.zeros((npad, 3), jnp.float32)], axis=0)
    posp = jnp.concatenate([posp, jnp.zeros((NPAD, 5), jnp.float32)], axis=1)
    post = posp[:, :8].T
    bpad = jnp.concatenate(
        [batch.astype(jnp.int32), jnp.full((npad,), 16, jnp.int32)])
    brow = bpad.reshape(1, NPAD)
    bcol = bpad.reshape(NPAD, 1)
    zcol = jnp.concatenate(
        [z.astype(jnp.int32), jnp.zeros((npad,), jnp.int32)]).reshape(NPAD, 1)
    embp = jnp.concatenate([emb, jnp.zeros((4, D), jnp.float32)], axis=0)

    rows = jnp.arange(H * C)
    hd = rows // C
    sel = (hd[:, None] == jnp.arange(H)[None, :]).astype(jnp.float32)
    aas = [jnp.concatenate([sel * asrc[b].reshape(-1)[:, None],
                            sel * adst[b].reshape(-1)[:, None]], axis=1)
           for b in range(NB)]

    p4, seg = _prep_call(post, brow)
    idx_sc = _sc_topk(p4, seg)
    idx_tc = _nbr_call(posp, post, brow, bcol)
    idx = jnp.concatenate([idx_sc, idx_tc], axis=0)

    h, table, ald = _pre_call(zcol, embp, Wc[0], aas[0])
    for b in range(NB):
        o = _sc_gat(table, idx, ald)
        if b < NB - 1:
            h, table, ald = _mid_call(
                o, h, bc[b].reshape(1, -1), g[b].reshape(1, -1),
                be[b].reshape(1, -1), Wf[b], bf[b].reshape(1, -1),
                Wc[b + 1], aas[b + 1])
        else:
            h = _post_call(
                o, h, bc[b].reshape(1, -1), g[b].reshape(1, -1),
                be[b].reshape(1, -1), Wf[b], bf[b].reshape(1, -1))
    return h[:N]
